# Initial kernel scaffold; baseline (speedup 1.0000x reference)
import jax
import jax.numpy as jnp
from jax.experimental import pallas as pl


def kernel(x, edge_index, edge_weight, params):
    raise NotImplementedError("write your pallas kernel here")


# speedup vs baseline: 17.4028x; 17.4028x over previous
import functools
import math

import jax
import jax.numpy as jnp
from jax import lax
from jax.experimental import pallas as pl
from jax.experimental.pallas import tpu as pltpu
from jax.experimental.pallas import tpu_sc as plsc

N_NODES = 512
E_EDGES = 8192
N_BATCH = 32
T_WIN = 10
C_IN = 16
C_HID = 64
FC_H = 256
FLATD = 2 * N_NODES * C_HID
FC2_OUT = N_NODES * N_NODES

_NC = 2
_NS = 16
_EPW = E_EDGES // (_NC * _NS)


def _sc_body(ei_ref, ew_ref, bp_out, degp_out,
             rbuf, cbuf, wbuf, idx2, val2, degidx2, zb, shB, shDeg):
    c = lax.axis_index("c")
    s = lax.axis_index("s")
    base = (c * _NS + s) * _EPW

    pltpu.sync_copy(ei_ref.at[0, pl.ds(base, _EPW)], rbuf)
    pltpu.sync_copy(ei_ref.at[1, pl.ds(base, _EPW)], cbuf)
    pltpu.sync_copy(ew_ref.at[pl.ds(base, _EPW)], wbuf)

    for k in range(32):
        zb[pl.ds(k * 16, 16)] = jnp.zeros((16,), jnp.float32)
    for k in range(32):
        pltpu.sync_copy(zb, shB.at[pl.ds((s * 32 + k) * 512, 512)])

    @pl.when(s == 0)
    def _():
        pltpu.sync_copy(zb, shDeg)

    plsc.subcore_barrier()

    for k in range(_EPW // 16):
        sl = pl.ds(k * 16, 16)
        r = rbuf[sl]
        cc = cbuf[sl]
        wv = wbuf[sl]
        wm = jnp.where(r == cc, jnp.zeros((16,), jnp.float32), wv)
        j, kk = divmod(k, 8)
        dsl = pl.ds(kk * 16, 16)
        idx2[j, dsl] = cc * N_NODES + r
        degidx2[j, dsl] = r
        val2[j, dsl] = wm

    for j in range(_EPW // 128):
        pltpu.sync_copy(val2.at[j], shB.at[idx2.at[j]], add=True)
        pltpu.sync_copy(val2.at[j], shDeg.at[degidx2.at[j]], add=True)

    plsc.subcore_barrier()

    seg = (N_NODES * N_NODES) // _NS
    pltpu.sync_copy(shB.at[pl.ds(s * seg, seg)],
                    bp_out.at[c, pl.ds(s * seg, seg)])

    @pl.when(s == 0)
    def _():
        pltpu.sync_copy(shDeg, degp_out.at[c])


def _sc_build(edge_index, edge_weight):
    mesh = plsc.VectorSubcoreMesh(core_axis_name="c", subcore_axis_name="s")
    f = pl.kernel(
        _sc_body,
        out_type=(
            jax.ShapeDtypeStruct((_NC, N_NODES * N_NODES), jnp.float32),
            jax.ShapeDtypeStruct((_NC, N_NODES), jnp.float32),
        ),
        mesh=mesh,
        scratch_types=[
            pltpu.VMEM((_EPW,), jnp.int32),
            pltpu.VMEM((_EPW,), jnp.int32),
            pltpu.VMEM((_EPW,), jnp.float32),
            pltpu.VMEM((_EPW // 128, 128), jnp.int32),
            pltpu.VMEM((_EPW // 128, 128), jnp.float32),
            pltpu.VMEM((_EPW // 128, 128), jnp.int32),
            pltpu.VMEM((N_NODES,), jnp.float32),
            pltpu.VMEM_SHARED((N_NODES * N_NODES,), jnp.float32),
            pltpu.VMEM_SHARED((N_NODES,), jnp.float32),
        ],
    )
    return f(edge_index, edge_weight)


def _fin_body(bp_ref, degc_ref, degr_ref, a_ref):
    bsum = bp_ref[0] + bp_ref[1]
    dc = degc_ref[0] + degc_ref[1]
    dr = degr_ref[0] + degr_ref[1]
    disc = jnp.where(dc > 0, lax.rsqrt(dc), 0.0)
    disr = jnp.where(dr > 0, lax.rsqrt(dr), 0.0)
    a_ref[...] = -(disc * disr) * bsum


def _finalize_a(bp, degp):
    return pl.pallas_call(
        _fin_body,
        out_shape=jax.ShapeDtypeStruct((N_NODES, N_NODES), jnp.float32),
    )(bp.reshape(_NC, N_NODES, N_NODES),
      degp.reshape(_NC, N_NODES, 1),
      degp.reshape(_NC, 1, N_NODES))


def _tconv(hflat, t_in, cin, wc, bc):
    t_out = t_in - 2
    rows = t_out * N_NODES
    cat = jnp.concatenate(
        [hflat[d * N_NODES:d * N_NODES + rows] for d in range(3)], axis=1)
    y = lax.dot_general(cat, wc, (((1,), (0,)), ((), ()))) + bc
    co = wc.shape[1] // 3
    a, g, c3 = y[:, :co], y[:, co:2 * co], y[:, 2 * co:]
    return jax.nn.relu(a * jax.nn.sigmoid(g) + c3)


def _cheb(hflat, t, a_mat, w0t, w1t, cb):
    p = jnp.concatenate(
        [lax.dot_general(a_mat, hflat[i * N_NODES:(i + 1) * N_NODES],
                         (((1,), (0,)), ((), ()))) for i in range(t)], axis=0)
    return jax.nn.relu(
        lax.dot_general(hflat, w0t, (((1,), (0,)), ((), ())))
        + lax.dot_general(p, w1t, (((1,), (0,)), ((), ())))
        + cb)


def _bn_relu(hflat, t, sc, bi):
    h3 = hflat.reshape(t, N_NODES, C_HID)
    h3 = jax.nn.relu(h3 * sc[None] + bi[None])
    return h3.reshape(t * N_NODES, C_HID)


def _st_body(x_ref, a_ref,
             w1c_ref, b1c_ref, w0t1_ref, w1t1_ref, cb1_ref, w2c_ref, b2c_ref,
             s1_ref, bb1_ref,
             w3c_ref, b3c_ref, w0t2_ref, w1t2_ref, cb2_ref, w4c_ref, b4c_ref,
             s2_ref, bb2_ref,
             out_ref):
    x = x_ref[0].reshape(T_WIN * N_NODES, C_IN)
    a_mat = a_ref[...]

    h = _tconv(x, T_WIN, C_IN, w1c_ref[...], b1c_ref[...])
    h = _cheb(h, 8, a_mat, w0t1_ref[...], w1t1_ref[...], cb1_ref[...])
    h = _tconv(h, 8, C_HID, w2c_ref[...], b2c_ref[...])
    h = _bn_relu(h, 6, s1_ref[...], bb1_ref[...])

    h = _tconv(h, 6, C_HID, w3c_ref[...], b3c_ref[...])
    h = _cheb(h, 4, a_mat, w0t2_ref[...], w1t2_ref[...], cb2_ref[...])
    h = _tconv(h, 4, C_HID, w4c_ref[...], b4c_ref[...])
    h = _bn_relu(h, 2, s2_ref[...], bb2_ref[...])

    out_ref[0] = h.reshape(2, N_NODES, C_HID)


def _stack_tconv_w(p, pref):
    ws = []
    bs = []
    for i in (1, 2, 3):
        k = p[pref + "_k%d" % i]
        w = jnp.transpose(k[:, :, 0, :], (2, 1, 0))
        ws.append(w.reshape(-1, k.shape[0]))
        bs.append(p[pref + "_b%d" % i])
    return jnp.concatenate(ws, axis=1), jnp.concatenate(bs)[None, :]


def _run_st(x, a_mat, p):
    w1c, b1c = _stack_tconv_w(p, "s1t1")
    w2c, b2c = _stack_tconv_w(p, "s1t2")
    w3c, b3c = _stack_tconv_w(p, "s2t1")
    w4c, b4c = _stack_tconv_w(p, "s2t2")
    bnscale = jnp.float32(1.0 / math.sqrt(1.0 + 1e-5))
    args = [
        x, a_mat,
        w1c, b1c, p["s1_chebW0"].T, p["s1_chebW1"].T, p["s1_chebb"][None, :],
        w2c, b2c,
        (p["bn1_g"] * bnscale)[:, None], p["bn1_b"][:, None],
        w3c, b3c, p["s2_chebW0"].T, p["s2_chebW1"].T, p["s2_chebb"][None, :],
        w4c, b4c,
        (p["bn2_g"] * bnscale)[:, None], p["bn2_b"][:, None],
    ]
    in_specs = [pl.BlockSpec((1,) + x.shape[1:], lambda b: (b, 0, 0, 0))]
    for t in args[1:]:
        nd = t.ndim
        in_specs.append(pl.BlockSpec(t.shape, functools.partial(
            lambda n, b: (0,) * n, nd)))
    return pl.pallas_call(
        _st_body,
        grid=(N_BATCH,),
        in_specs=in_specs,
        out_specs=pl.BlockSpec((1, 2, N_NODES, C_HID), lambda b: (b, 0, 0, 0)),
        out_shape=jax.ShapeDtypeStruct((N_BATCH, 2, N_NODES, C_HID),
                                       jnp.float32),
    )(*args)


_FC1_KB = 8192
_FC2_RB = 8192


def _fc1_body(h_ref, w_ref, b_ref, o_ref):
    k = pl.program_id(0)

    @pl.when(k == 0)
    def _():
        o_ref[...] = jnp.zeros_like(o_ref)

    o_ref[...] += lax.dot_general(h_ref[...], w_ref[...],
                                  (((1,), (1,)), ((), ())))

    @pl.when(k == pl.num_programs(0) - 1)
    def _():
        o_ref[...] = jax.nn.relu(o_ref[...] + b_ref[...])


def _run_fc1(h, w, b):
    nk = FLATD // _FC1_KB
    return pl.pallas_call(
        _fc1_body,
        grid=(nk,),
        in_specs=[
            pl.BlockSpec((N_BATCH, _FC1_KB), lambda k: (0, k)),
            pl.BlockSpec((FC_H, _FC1_KB), lambda k: (0, k)),
            pl.BlockSpec((1, FC_H), lambda k: (0, 0)),
        ],
        out_specs=pl.BlockSpec((N_BATCH, FC_H), lambda k: (0, 0)),
        out_shape=jax.ShapeDtypeStruct((N_BATCH, FC_H), jnp.float32),
    )(h, w, b[None, :])


def _fc2_body(h_ref, w_ref, b_ref, o_ref):
    y = lax.dot_general(h_ref[...], w_ref[...], (((1,), (1,)), ((), ())))
    o_ref[...] = jax.nn.sigmoid(y + b_ref[...])


def _run_fc2(h, w, b):
    nj = FC2_OUT // _FC2_RB
    return pl.pallas_call(
        _fc2_body,
        grid=(nj,),
        in_specs=[
            pl.BlockSpec((N_BATCH, FC_H), lambda j: (0, 0)),
            pl.BlockSpec((_FC2_RB, FC_H), lambda j: (j, 0)),
            pl.BlockSpec((1, _FC2_RB), lambda j: (0, j)),
        ],
        out_specs=pl.BlockSpec((N_BATCH, _FC2_RB), lambda j: (0, j)),
        out_shape=jax.ShapeDtypeStruct((N_BATCH, FC2_OUT), jnp.float32),
    )(h, w, b[None, :])


def kernel(x, edge_index, edge_weight, params):
    bp, degp = _sc_build(edge_index.astype(jnp.int32),
                         edge_weight.astype(jnp.float32))
    a_mat = _finalize_a(bp, degp)
    h = _run_st(x, a_mat, params)
    hmid = _run_fc1(h.reshape(N_BATCH, FLATD), params["fc1_W"], params["fc1_b"])
    out = _run_fc2(hmid, params["fc2_W"], params["fc2_b"])
    return out.reshape(N_BATCH, N_NODES, N_NODES)


# speedup vs baseline: 17.7096x; 1.0176x over previous
import functools
import math

import jax
import jax.numpy as jnp
from jax import lax
from jax.experimental import pallas as pl
from jax.experimental.pallas import tpu as pltpu
from jax.experimental.pallas import tpu_sc as plsc

N_NODES = 512
E_EDGES = 8192
N_BATCH = 32
T_WIN = 10
C_IN = 16
C_HID = 64
FC_H = 256
FLATD = 2 * N_NODES * C_HID
FC2_OUT = N_NODES * N_NODES

_NC = 2
_NS = 16
_EPW = E_EDGES // (_NC * _NS)


def _sc_body(ei_ref, ew_ref, bp_out, degp_out,
             rbuf, cbuf, wbuf, idx2, val2, degidx2, zb, shB, shDeg):
    c = lax.axis_index("c")
    s = lax.axis_index("s")
    base = (c * _NS + s) * _EPW

    pltpu.sync_copy(ei_ref.at[0, pl.ds(base, _EPW)], rbuf)
    pltpu.sync_copy(ei_ref.at[1, pl.ds(base, _EPW)], cbuf)
    pltpu.sync_copy(ew_ref.at[pl.ds(base, _EPW)], wbuf)

    for k in range(32):
        zb[pl.ds(k * 16, 16)] = jnp.zeros((16,), jnp.float32)
    for k in range(32):
        pltpu.sync_copy(zb, shB.at[pl.ds((s * 32 + k) * 512, 512)])

    @pl.when(s == 0)
    def _():
        pltpu.sync_copy(zb, shDeg)

    plsc.subcore_barrier()

    for k in range(_EPW // 16):
        sl = pl.ds(k * 16, 16)
        r = rbuf[sl]
        cc = cbuf[sl]
        wv = wbuf[sl]
        wm = jnp.where(r == cc, jnp.zeros((16,), jnp.float32), wv)
        j, kk = divmod(k, 8)
        dsl = pl.ds(kk * 16, 16)
        idx2[j, dsl] = cc * N_NODES + r
        degidx2[j, dsl] = r
        val2[j, dsl] = wm

    for j in range(_EPW // 128):
        pltpu.sync_copy(val2.at[j], shB.at[idx2.at[j]], add=True)
        pltpu.sync_copy(val2.at[j], shDeg.at[degidx2.at[j]], add=True)

    plsc.subcore_barrier()

    seg = (N_NODES * N_NODES) // _NS
    base_b = c * (N_NODES * N_NODES) + s * seg
    pltpu.sync_copy(shB.at[pl.ds(s * seg, seg)], bp_out.at[pl.ds(base_b, seg)])

    @pl.when(s == 0)
    def _():
        pltpu.sync_copy(shDeg, degp_out.at[pl.ds(c * N_NODES, N_NODES)])


def _sc_build(edge_index, edge_weight):
    mesh = plsc.VectorSubcoreMesh(core_axis_name="c", subcore_axis_name="s")
    f = pl.kernel(
        _sc_body,
        out_type=(
            jax.ShapeDtypeStruct((_NC * N_NODES * N_NODES,), jnp.float32),
            jax.ShapeDtypeStruct((_NC * N_NODES,), jnp.float32),
        ),
        mesh=mesh,
        scratch_types=[
            pltpu.VMEM((_EPW,), jnp.int32),
            pltpu.VMEM((_EPW,), jnp.int32),
            pltpu.VMEM((_EPW,), jnp.float32),
            pltpu.VMEM((_EPW // 128, 128), jnp.int32),
            pltpu.VMEM((_EPW // 128, 128), jnp.float32),
            pltpu.VMEM((_EPW // 128, 128), jnp.int32),
            pltpu.VMEM((N_NODES,), jnp.float32),
            pltpu.VMEM_SHARED((N_NODES * N_NODES,), jnp.float32),
            pltpu.VMEM_SHARED((N_NODES,), jnp.float32),
        ],
    )
    return f(edge_index, edge_weight)


def _tconv(hflat, t_in, cin, wc, bc):
    t_out = t_in - 2
    rows = t_out * N_NODES
    cat = jnp.concatenate(
        [hflat[d * N_NODES:d * N_NODES + rows] for d in range(3)], axis=1)
    y = lax.dot_general(cat, wc, (((1,), (0,)), ((), ()))) + bc
    co = wc.shape[1] // 3
    a, g, c3 = y[:, :co], y[:, co:2 * co], y[:, 2 * co:]
    return jax.nn.relu(a * jax.nn.sigmoid(g) + c3)


def _cheb(hflat, t, a_mat, bd0, bd1, cbt):
    hcat = jnp.concatenate(
        [hflat[i * N_NODES:(i + 1) * N_NODES] for i in range(t)], axis=1)
    p = lax.dot_general(a_mat, hcat, (((1,), (0,)), ((), ())))
    g = jax.nn.relu(
        lax.dot_general(hcat, bd0, (((1,), (0,)), ((), ())))
        + lax.dot_general(p, bd1, (((1,), (0,)), ((), ())))
        + cbt)
    return jnp.concatenate(
        [g[:, i * C_HID:(i + 1) * C_HID] for i in range(t)], axis=0)


def _bn_relu(hflat, t, sc, bi):
    h3 = hflat.reshape(t, N_NODES, C_HID)
    h3 = jax.nn.relu(h3 * sc[None] + bi[None])
    return h3.reshape(t * N_NODES, C_HID)


def _st_body(x_ref, bp_ref, degc_ref, degr_ref,
             w1c_ref, b1c_ref, bd01_ref, bd11_ref, cb1_ref, w2c_ref, b2c_ref,
             s1_ref, bb1_ref,
             w3c_ref, b3c_ref, bd02_ref, bd12_ref, cb2_ref, w4c_ref, b4c_ref,
             s2_ref, bb2_ref,
             out_ref, a_scr):
    @pl.when(pl.program_id(0) == 0)
    def _():
        dc = degc_ref[0] + degc_ref[1]
        dr = degr_ref[0] + degr_ref[1]
        disc = jnp.where(dc > 0, lax.rsqrt(dc), 0.0)
        disr = jnp.where(dr > 0, lax.rsqrt(dr), 0.0)
        a_scr[...] = -(disc * disr) * (bp_ref[0] + bp_ref[1])

    x = x_ref[0].reshape(T_WIN * N_NODES, C_IN)
    a_mat = a_scr[...]

    h = _tconv(x, T_WIN, C_IN, w1c_ref[...], b1c_ref[...])
    h = _cheb(h, 8, a_mat, bd01_ref[...], bd11_ref[...], cb1_ref[...])
    h = _tconv(h, 8, C_HID, w2c_ref[...], b2c_ref[...])
    h = _bn_relu(h, 6, s1_ref[...], bb1_ref[...])

    h = _tconv(h, 6, C_HID, w3c_ref[...], b3c_ref[...])
    h = _cheb(h, 4, a_mat, bd02_ref[...], bd12_ref[...], cb2_ref[...])
    h = _tconv(h, 4, C_HID, w4c_ref[...], b4c_ref[...])
    h = _bn_relu(h, 2, s2_ref[...], bb2_ref[...])

    out_ref[0] = h.reshape(2, N_NODES, C_HID)


def _stack_tconv_w(p, pref):
    ws = []
    bs = []
    for i in (1, 2, 3):
        k = p[pref + "_k%d" % i]
        w = jnp.transpose(k[:, :, 0, :], (2, 1, 0))
        ws.append(w.reshape(-1, k.shape[0]))
        bs.append(p[pref + "_b%d" % i])
    return jnp.concatenate(ws, axis=1), jnp.concatenate(bs)[None, :]


def _run_st(x, bp, degp, p):
    w1c, b1c = _stack_tconv_w(p, "s1t1")
    w2c, b2c = _stack_tconv_w(p, "s1t2")
    w3c, b3c = _stack_tconv_w(p, "s2t1")
    w4c, b4c = _stack_tconv_w(p, "s2t2")
    bnscale = jnp.float32(1.0 / math.sqrt(1.0 + 1e-5))
    eye8 = jnp.eye(8, dtype=jnp.float32)
    eye4 = jnp.eye(4, dtype=jnp.float32)
    args = [
        x,
        bp.reshape(_NC, N_NODES, N_NODES),
        degp.reshape(_NC, N_NODES, 1),
        degp.reshape(_NC, 1, N_NODES),
        w1c, b1c,
        jnp.kron(eye8, p["s1_chebW0"].T), jnp.kron(eye8, p["s1_chebW1"].T),
        jnp.tile(p["s1_chebb"][None, :], (1, 8)),
        w2c, b2c,
        (p["bn1_g"] * bnscale)[:, None], p["bn1_b"][:, None],
        w3c, b3c,
        jnp.kron(eye4, p["s2_chebW0"].T), jnp.kron(eye4, p["s2_chebW1"].T),
        jnp.tile(p["s2_chebb"][None, :], (1, 4)),
        w4c, b4c,
        (p["bn2_g"] * bnscale)[:, None], p["bn2_b"][:, None],
    ]
    in_specs = [pl.BlockSpec((1,) + x.shape[1:], lambda b: (b, 0, 0, 0))]
    for t in args[1:]:
        nd = t.ndim
        in_specs.append(pl.BlockSpec(t.shape, functools.partial(
            lambda n, b: (0,) * n, nd)))
    return pl.pallas_call(
        _st_body,
        grid=(N_BATCH,),
        in_specs=in_specs,
        out_specs=pl.BlockSpec((1, 2, N_NODES, C_HID), lambda b: (b, 0, 0, 0)),
        out_shape=jax.ShapeDtypeStruct((N_BATCH, 2, N_NODES, C_HID),
                                       jnp.float32),
        scratch_shapes=[pltpu.VMEM((N_NODES, N_NODES), jnp.float32)],
    )(*args)


_FC1_KB = 8192
_FC2_RB = 8192
_NK1 = FLATD // _FC1_KB
_NJ2 = FC2_OUT // _FC2_RB


def _fc_body(h_ref, w1_ref, b1_ref, w2_ref, b2_ref, o_ref, hm_scr):
    i = pl.program_id(0)

    @pl.when(i == 0)
    def _():
        hm_scr[...] = jnp.zeros_like(hm_scr)

    @pl.when(i < _NK1)
    def _():
        hm_scr[...] += lax.dot_general(h_ref[...], w1_ref[...],
                                       (((1,), (1,)), ((), ())))

    @pl.when(i >= _NK1)
    def _():
        act = jax.nn.relu(hm_scr[...] + b1_ref[...])
        y = lax.dot_general(act, w2_ref[...], (((1,), (1,)), ((), ())))
        o_ref[...] = jax.nn.sigmoid(y + b2_ref[...])


def _run_fc(h, w1, b1, w2, b2):
    def clip(i, lo, hi):
        return jnp.minimum(jnp.maximum(i + lo, 0), hi)
    return pl.pallas_call(
        _fc_body,
        grid=(_NK1 + _NJ2,),
        in_specs=[
            pl.BlockSpec((N_BATCH, _FC1_KB), lambda i: (0, clip(i, 0, _NK1 - 1))),
            pl.BlockSpec((FC_H, _FC1_KB), lambda i: (0, clip(i, 0, _NK1 - 1))),
            pl.BlockSpec((1, FC_H), lambda i: (0, 0)),
            pl.BlockSpec((_FC2_RB, FC_H), lambda i: (clip(i, -_NK1, _NJ2 - 1), 0)),
            pl.BlockSpec((1, _FC2_RB), lambda i: (0, clip(i, -_NK1, _NJ2 - 1))),
        ],
        out_specs=pl.BlockSpec((N_BATCH, _FC2_RB),
                               lambda i: (0, clip(i, -_NK1, _NJ2 - 1))),
        out_shape=jax.ShapeDtypeStruct((N_BATCH, FC2_OUT), jnp.float32),
        scratch_shapes=[pltpu.VMEM((N_BATCH, FC_H), jnp.float32)],
    )(h, w1, b1[None, :], w2, b2[None, :])


def kernel(x, edge_index, edge_weight, params):
    bp, degp = _sc_build(edge_index.astype(jnp.int32),
                         edge_weight.astype(jnp.float32))
    h = _run_st(x, bp, degp, params)
    out = _run_fc(h.reshape(N_BATCH, FLATD), params["fc1_W"], params["fc1_b"],
                  params["fc2_W"], params["fc2_b"])
    return out.reshape(N_BATCH, N_NODES, N_NODES)


# speedup vs baseline: 18.8117x; 1.0622x over previous
import functools
import math

import jax
import jax.numpy as jnp
from jax import lax
from jax.experimental import pallas as pl
from jax.experimental.pallas import tpu as pltpu
from jax.experimental.pallas import tpu_sc as plsc

N_NODES = 512
E_EDGES = 8192
N_BATCH = 32
T_WIN = 10
C_IN = 16
C_HID = 64
FC_H = 256
FLATD = 2 * N_NODES * C_HID
FC2_OUT = N_NODES * N_NODES

_NC = 2
_NS = 16
_EPW = E_EDGES // (_NC * _NS)


def _sc_body(ei_ref, ew_ref, bp_out, degp_out,
             rbuf, cbuf, wbuf, idx2, val2, degidx2, zb, shB, shDeg):
    c = lax.axis_index("c")
    s = lax.axis_index("s")
    base = (c * _NS + s) * _EPW

    pltpu.sync_copy(ei_ref.at[0, pl.ds(base, _EPW)], rbuf)
    pltpu.sync_copy(ei_ref.at[1, pl.ds(base, _EPW)], cbuf)
    pltpu.sync_copy(ew_ref.at[pl.ds(base, _EPW)], wbuf)

    for k in range(32):
        zb[pl.ds(k * 16, 16)] = jnp.zeros((16,), jnp.float32)
    for k in range(32):
        pltpu.sync_copy(zb, shB.at[pl.ds((s * 32 + k) * 512, 512)])

    @pl.when(s == 0)
    def _():
        pltpu.sync_copy(zb, shDeg)

    plsc.subcore_barrier()

    for k in range(_EPW // 16):
        sl = pl.ds(k * 16, 16)
        r = rbuf[sl]
        cc = cbuf[sl]
        wv = wbuf[sl]
        wm = jnp.where(r == cc, jnp.zeros((16,), jnp.float32), wv)
        j, kk = divmod(k, 8)
        dsl = pl.ds(kk * 16, 16)
        idx2[j, dsl] = cc * N_NODES + r
        degidx2[j, dsl] = r
        val2[j, dsl] = wm

    for j in range(_EPW // 128):
        pltpu.sync_copy(val2.at[j], shB.at[idx2.at[j]], add=True)
        pltpu.sync_copy(val2.at[j], shDeg.at[degidx2.at[j]], add=True)

    plsc.subcore_barrier()

    seg = (N_NODES * N_NODES) // _NS
    base_b = c * (N_NODES * N_NODES) + s * seg
    pltpu.sync_copy(shB.at[pl.ds(s * seg, seg)], bp_out.at[pl.ds(base_b, seg)])

    @pl.when(s == 0)
    def _():
        pltpu.sync_copy(shDeg, degp_out.at[pl.ds(c * N_NODES, N_NODES)])


def _sc_build(edge_index, edge_weight):
    mesh = plsc.VectorSubcoreMesh(core_axis_name="c", subcore_axis_name="s")
    f = pl.kernel(
        _sc_body,
        out_type=(
            jax.ShapeDtypeStruct((_NC * N_NODES * N_NODES,), jnp.float32),
            jax.ShapeDtypeStruct((_NC * N_NODES,), jnp.float32),
        ),
        mesh=mesh,
        scratch_types=[
            pltpu.VMEM((_EPW,), jnp.int32),
            pltpu.VMEM((_EPW,), jnp.int32),
            pltpu.VMEM((_EPW,), jnp.float32),
            pltpu.VMEM((_EPW // 128, 128), jnp.int32),
            pltpu.VMEM((_EPW // 128, 128), jnp.float32),
            pltpu.VMEM((_EPW // 128, 128), jnp.int32),
            pltpu.VMEM((N_NODES,), jnp.float32),
            pltpu.VMEM_SHARED((N_NODES * N_NODES,), jnp.float32),
            pltpu.VMEM_SHARED((N_NODES,), jnp.float32),
        ],
    )
    return f(edge_index, edge_weight)


def _tconv(hflat, t_in, cin, wc, bc):
    t_out = t_in - 2
    rows = t_out * N_NODES
    cat = jnp.concatenate(
        [hflat[d * N_NODES:d * N_NODES + rows] for d in range(3)], axis=1)
    y = lax.dot_general(cat, wc, (((1,), (0,)), ((), ()))) + bc
    co = wc.shape[1] // 3
    a, g, c3 = y[:, :co], y[:, co:2 * co], y[:, 2 * co:]
    return jax.nn.relu(a * jax.nn.sigmoid(g) + c3)


def _cheb(hflat, t, a_mat, bd0, bd1, cbt):
    hcat = jnp.concatenate(
        [hflat[i * N_NODES:(i + 1) * N_NODES] for i in range(t)], axis=1)
    p = lax.dot_general(a_mat, hcat, (((1,), (0,)), ((), ())))
    g = jax.nn.relu(
        lax.dot_general(hcat, bd0, (((1,), (0,)), ((), ())))
        + lax.dot_general(p, bd1, (((1,), (0,)), ((), ())))
        + cbt)
    return jnp.concatenate(
        [g[:, i * C_HID:(i + 1) * C_HID] for i in range(t)], axis=0)


def _bn_relu(hflat, t, sc, bi):
    h3 = hflat.reshape(t, N_NODES, C_HID)
    h3 = jax.nn.relu(h3 * sc[None] + bi[None])
    return h3.reshape(t * N_NODES, C_HID)


def _st_body(x_ref, bp_ref, degc_ref, degr_ref,
             w1c_ref, b1c_ref, bd01_ref, bd11_ref, cb1_ref, w2c_ref, b2c_ref,
             s1_ref, bb1_ref,
             w3c_ref, b3c_ref, bd02_ref, bd12_ref, cb2_ref, w4c_ref, b4c_ref,
             s2_ref, bb2_ref,
             out_ref, a_scr):
    @pl.when(pl.program_id(0) == 0)
    def _():
        dc = degc_ref[0] + degc_ref[1]
        dr = degr_ref[0] + degr_ref[1]
        disc = jnp.where(dc > 0, lax.rsqrt(dc), 0.0)
        disr = jnp.where(dr > 0, lax.rsqrt(dr), 0.0)
        a_scr[...] = -(disc * disr) * (bp_ref[0] + bp_ref[1])

    x = x_ref[0].reshape(T_WIN * N_NODES, C_IN)
    a_mat = a_scr[...]

    h = _tconv(x, T_WIN, C_IN, w1c_ref[...], b1c_ref[...])
    h = _cheb(h, 8, a_mat, bd01_ref[...], bd11_ref[...], cb1_ref[...])
    h = _tconv(h, 8, C_HID, w2c_ref[...], b2c_ref[...])
    h = _bn_relu(h, 6, s1_ref[...], bb1_ref[...])

    h = _tconv(h, 6, C_HID, w3c_ref[...], b3c_ref[...])
    h = _cheb(h, 4, a_mat, bd02_ref[...], bd12_ref[...], cb2_ref[...])
    h = _tconv(h, 4, C_HID, w4c_ref[...], b4c_ref[...])
    h = _bn_relu(h, 2, s2_ref[...], bb2_ref[...])

    out_ref[0] = h.reshape(2, N_NODES, C_HID)


def _stack_tconv_w(p, pref):
    ws = []
    bs = []
    for i in (1, 2, 3):
        k = p[pref + "_k%d" % i]
        w = jnp.transpose(k[:, :, 0, :], (2, 1, 0))
        ws.append(w.reshape(-1, k.shape[0]))
        bs.append(p[pref + "_b%d" % i])
    return jnp.concatenate(ws, axis=1), jnp.concatenate(bs)[None, :]


def _run_st(x, bp, degp, p):
    w1c, b1c = _stack_tconv_w(p, "s1t1")
    w2c, b2c = _stack_tconv_w(p, "s1t2")
    w3c, b3c = _stack_tconv_w(p, "s2t1")
    w4c, b4c = _stack_tconv_w(p, "s2t2")
    bnscale = jnp.float32(1.0 / math.sqrt(1.0 + 1e-5))
    eye8 = jnp.eye(8, dtype=jnp.float32)
    eye4 = jnp.eye(4, dtype=jnp.float32)
    args = [
        x,
        bp.reshape(_NC, N_NODES, N_NODES),
        degp.reshape(_NC, N_NODES, 1),
        degp.reshape(_NC, 1, N_NODES),
        w1c, b1c,
        jnp.kron(eye8, p["s1_chebW0"].T), jnp.kron(eye8, p["s1_chebW1"].T),
        jnp.tile(p["s1_chebb"][None, :], (1, 8)),
        w2c, b2c,
        (p["bn1_g"] * bnscale)[:, None], p["bn1_b"][:, None],
        w3c, b3c,
        jnp.kron(eye4, p["s2_chebW0"].T), jnp.kron(eye4, p["s2_chebW1"].T),
        jnp.tile(p["s2_chebb"][None, :], (1, 4)),
        w4c, b4c,
        (p["bn2_g"] * bnscale)[:, None], p["bn2_b"][:, None],
    ]
    in_specs = [pl.BlockSpec((1,) + x.shape[1:], lambda b: (b, 0, 0, 0))]
    for t in args[1:]:
        nd = t.ndim
        in_specs.append(pl.BlockSpec(t.shape, functools.partial(
            lambda n, b: (0,) * n, nd)))
    return pl.pallas_call(
        _st_body,
        grid=(N_BATCH,),
        in_specs=in_specs,
        out_specs=pl.BlockSpec((1, 2, N_NODES, C_HID), lambda b: (b, 0, 0, 0)),
        out_shape=jax.ShapeDtypeStruct((N_BATCH, 2, N_NODES, C_HID),
                                       jnp.float32),
        scratch_shapes=[pltpu.VMEM((N_NODES, N_NODES), jnp.float32)],
    )(*args)


_FC1_KB = 8192
_FC2_RB = 8192
_NK1 = FLATD // _FC1_KB
_NJ2 = FC2_OUT // _FC2_RB


def _fc_body(h_ref, w1_ref, b1_ref, w2_ref, b2_ref, o_ref, hm_scr):
    i = pl.program_id(0)

    @pl.when(i == 0)
    def _():
        hm_scr[...] = jnp.zeros_like(hm_scr)

    @pl.when(i < _NK1)
    def _():
        hm_scr[...] += lax.dot_general(h_ref[...], w1_ref[...],
                                       (((1,), (1,)), ((), ())))

    @pl.when(i >= _NK1)
    def _():
        act = jax.nn.relu(hm_scr[...] + b1_ref[...])
        y = lax.dot_general(act, w2_ref[...], (((1,), (1,)), ((), ())))
        o_ref[...] = jax.nn.sigmoid(y + b2_ref[...]).reshape(
            N_BATCH, _FC2_RB // N_NODES, N_NODES)


def _run_fc(h, w1, b1, w2, b2):
    def clip(i, lo, hi):
        return jnp.minimum(jnp.maximum(i + lo, 0), hi)
    return pl.pallas_call(
        _fc_body,
        grid=(_NK1 + _NJ2,),
        in_specs=[
            pl.BlockSpec((N_BATCH, _FC1_KB), lambda i: (0, clip(i, 0, _NK1 - 1))),
            pl.BlockSpec((FC_H, _FC1_KB), lambda i: (0, clip(i, 0, _NK1 - 1))),
            pl.BlockSpec((1, FC_H), lambda i: (0, 0)),
            pl.BlockSpec((_FC2_RB, FC_H), lambda i: (clip(i, -_NK1, _NJ2 - 1), 0)),
            pl.BlockSpec((1, _FC2_RB), lambda i: (0, clip(i, -_NK1, _NJ2 - 1))),
        ],
        out_specs=pl.BlockSpec((N_BATCH, _FC2_RB // N_NODES, N_NODES),
                               lambda i: (0, clip(i, -_NK1, _NJ2 - 1), 0)),
        out_shape=jax.ShapeDtypeStruct((N_BATCH, N_NODES, N_NODES),
                                       jnp.float32),
        scratch_shapes=[pltpu.VMEM((N_BATCH, FC_H), jnp.float32)],
    )(h, w1, b1[None, :], w2, b2[None, :])


def kernel(x, edge_index, edge_weight, params):
    bp, degp = _sc_build(edge_index.astype(jnp.int32),
                         edge_weight.astype(jnp.float32))
    h = _run_st(x, bp, degp, params)
    return _run_fc(h.reshape(N_BATCH, FLATD), params["fc1_W"], params["fc1_b"],
                   params["fc2_W"], params["fc2_b"])


# speedup vs baseline: 19.3063x; 1.0263x over previous
import functools
import math

import jax
import jax.numpy as jnp
from jax import lax
from jax.experimental import pallas as pl
from jax.experimental.pallas import tpu as pltpu
from jax.experimental.pallas import tpu_sc as plsc

N_NODES = 512
E_EDGES = 8192
N_BATCH = 32
T_WIN = 10
C_IN = 16
C_HID = 64
FC_H = 256
FLATD = 2 * N_NODES * C_HID
FC2_OUT = N_NODES * N_NODES

_NC = 2
_NS = 16
_EPW = E_EDGES // (_NC * _NS)


def _sc_body(ei_ref, ew_ref, bp_out, degp_out,
             rbuf, cbuf, wbuf, idx2, val2, degidx2, zb, shB, shDeg):
    c = lax.axis_index("c")
    s = lax.axis_index("s")
    base = (c * _NS + s) * _EPW

    pltpu.sync_copy(ei_ref.at[0, pl.ds(base, _EPW)], rbuf)
    pltpu.sync_copy(ei_ref.at[1, pl.ds(base, _EPW)], cbuf)
    pltpu.sync_copy(ew_ref.at[pl.ds(base, _EPW)], wbuf)

    for k in range(32):
        zb[pl.ds(k * 16, 16)] = jnp.zeros((16,), jnp.float32)
    for k in range(32):
        pltpu.sync_copy(zb, shB.at[pl.ds((s * 32 + k) * 512, 512)])

    @pl.when(s == 0)
    def _():
        pltpu.sync_copy(zb, shDeg)

    plsc.subcore_barrier()

    for k in range(_EPW // 16):
        sl = pl.ds(k * 16, 16)
        r = rbuf[sl]
        cc = cbuf[sl]
        wv = wbuf[sl]
        wm = jnp.where(r == cc, jnp.zeros((16,), jnp.float32), wv)
        j, kk = divmod(k, 8)
        dsl = pl.ds(kk * 16, 16)
        idx2[j, dsl] = cc * N_NODES + r
        degidx2[j, dsl] = r
        val2[j, dsl] = wm

    for j in range(_EPW // 128):
        pltpu.sync_copy(val2.at[j], shB.at[idx2.at[j]], add=True)
        pltpu.sync_copy(val2.at[j], shDeg.at[degidx2.at[j]], add=True)

    plsc.subcore_barrier()

    seg = (N_NODES * N_NODES) // _NS
    base_b = c * (N_NODES * N_NODES) + s * seg
    pltpu.sync_copy(shB.at[pl.ds(s * seg, seg)], bp_out.at[pl.ds(base_b, seg)])

    @pl.when(s == 0)
    def _():
        pltpu.sync_copy(shDeg, degp_out.at[pl.ds(c * N_NODES, N_NODES)])


def _sc_build(edge_index, edge_weight):
    mesh = plsc.VectorSubcoreMesh(core_axis_name="c", subcore_axis_name="s")
    f = pl.kernel(
        _sc_body,
        out_type=(
            jax.ShapeDtypeStruct((_NC * N_NODES * N_NODES,), jnp.float32),
            jax.ShapeDtypeStruct((_NC * N_NODES,), jnp.float32),
        ),
        mesh=mesh,
        scratch_types=[
            pltpu.VMEM((_EPW,), jnp.int32),
            pltpu.VMEM((_EPW,), jnp.int32),
            pltpu.VMEM((_EPW,), jnp.float32),
            pltpu.VMEM((_EPW // 128, 128), jnp.int32),
            pltpu.VMEM((_EPW // 128, 128), jnp.float32),
            pltpu.VMEM((_EPW // 128, 128), jnp.int32),
            pltpu.VMEM((N_NODES,), jnp.float32),
            pltpu.VMEM_SHARED((N_NODES * N_NODES,), jnp.float32),
            pltpu.VMEM_SHARED((N_NODES,), jnp.float32),
        ],
    )
    return f(edge_index, edge_weight)


def _dotf(a, b):
    return lax.dot_general(a, b, (((1,), (0,)), ((), ())),
                           preferred_element_type=jnp.float32)


def _tconv(hflat, t_in, cin, wc, bc):
    t_out = t_in - 2
    rows = t_out * N_NODES
    cat = jnp.concatenate(
        [hflat[d * N_NODES:d * N_NODES + rows] for d in range(3)], axis=1)
    y = _dotf(cat, wc) + bc
    co = wc.shape[1] // 3
    a, g, c3 = y[:, :co], y[:, co:2 * co], y[:, 2 * co:]
    return jax.nn.relu(a * jax.nn.sigmoid(g) + c3)


def _cheb(hflat, t, a_mat, bd0, bd1, cbt):
    hcat = jnp.concatenate(
        [hflat[i * N_NODES:(i + 1) * N_NODES] for i in range(t)], axis=1)
    p = _bf(_dotf(a_mat, hcat))
    g = _bf(jax.nn.relu(_dotf(hcat, bd0) + _dotf(p, bd1) + cbt))
    return jnp.concatenate(
        [g[:, i * C_HID:(i + 1) * C_HID] for i in range(t)], axis=0)


def _bn_relu(hflat, t, sc, bi):
    h3 = hflat.reshape(t, N_NODES, C_HID)
    h3 = jax.nn.relu(h3 * sc[None] + bi[None])
    return h3.reshape(t * N_NODES, C_HID)


def _bf(v):
    return v.astype(jnp.bfloat16)


def _st_body(x_ref, bp_ref, degc_ref, degr_ref,
             w1c_ref, b1c_ref, bd01_ref, bd11_ref, cb1_ref, w2c_ref, b2c_ref,
             s1_ref, bb1_ref,
             w3c_ref, b3c_ref, bd02_ref, bd12_ref, cb2_ref, w4c_ref, b4c_ref,
             s2_ref, bb2_ref,
             out_ref, a_scr):
    @pl.when(pl.program_id(0) == 0)
    def _():
        dc = degc_ref[0] + degc_ref[1]
        dr = degr_ref[0] + degr_ref[1]
        disc = jnp.where(dc > 0, lax.rsqrt(dc), 0.0)
        disr = jnp.where(dr > 0, lax.rsqrt(dr), 0.0)
        a_scr[...] = _bf(-(disc * disr) * (bp_ref[0] + bp_ref[1]))

    x = _bf(x_ref[0].reshape(T_WIN * N_NODES, C_IN))
    a_mat = a_scr[...]

    h = _bf(_tconv(x, T_WIN, C_IN, w1c_ref[...], b1c_ref[...]))
    h = _cheb(h, 8, a_mat, bd01_ref[...], bd11_ref[...], cb1_ref[...])
    h = _bf(_tconv(h, 8, C_HID, w2c_ref[...], b2c_ref[...]))
    h = _bf(_bn_relu(h, 6, s1_ref[...], bb1_ref[...]))

    h = _bf(_tconv(h, 6, C_HID, w3c_ref[...], b3c_ref[...]))
    h = _cheb(h, 4, a_mat, bd02_ref[...], bd12_ref[...], cb2_ref[...])
    h = _bf(_tconv(h, 4, C_HID, w4c_ref[...], b4c_ref[...]))
    h = _bn_relu(h, 2, s2_ref[...], bb2_ref[...])

    out_ref[0] = h.reshape(2, N_NODES, C_HID).astype(jnp.float32)


def _stack_tconv_w(p, pref):
    ws = []
    bs = []
    for i in (1, 2, 3):
        k = p[pref + "_k%d" % i]
        w = jnp.transpose(k[:, :, 0, :], (2, 1, 0))
        ws.append(w.reshape(-1, k.shape[0]))
        bs.append(p[pref + "_b%d" % i])
    return jnp.concatenate(ws, axis=1), jnp.concatenate(bs)[None, :]


def _run_st(x, bp, degp, p):
    w1c, b1c = _stack_tconv_w(p, "s1t1")
    w2c, b2c = _stack_tconv_w(p, "s1t2")
    w3c, b3c = _stack_tconv_w(p, "s2t1")
    w4c, b4c = _stack_tconv_w(p, "s2t2")
    bnscale = jnp.float32(1.0 / math.sqrt(1.0 + 1e-5))
    eye8 = jnp.eye(8, dtype=jnp.float32)
    eye4 = jnp.eye(4, dtype=jnp.float32)
    args = [
        x,
        bp.reshape(_NC, N_NODES, N_NODES),
        degp.reshape(_NC, N_NODES, 1),
        degp.reshape(_NC, 1, N_NODES),
        _bf(w1c), b1c,
        _bf(jnp.kron(eye8, p["s1_chebW0"].T)),
        _bf(jnp.kron(eye8, p["s1_chebW1"].T)),
        jnp.tile(p["s1_chebb"][None, :], (1, 8)),
        _bf(w2c), b2c,
        (p["bn1_g"] * bnscale)[:, None], p["bn1_b"][:, None],
        _bf(w3c), b3c,
        _bf(jnp.kron(eye4, p["s2_chebW0"].T)),
        _bf(jnp.kron(eye4, p["s2_chebW1"].T)),
        jnp.tile(p["s2_chebb"][None, :], (1, 4)),
        _bf(w4c), b4c,
        (p["bn2_g"] * bnscale)[:, None], p["bn2_b"][:, None],
    ]
    in_specs = [pl.BlockSpec((1,) + x.shape[1:], lambda b: (b, 0, 0, 0))]
    for t in args[1:]:
        nd = t.ndim
        in_specs.append(pl.BlockSpec(t.shape, functools.partial(
            lambda n, b: (0,) * n, nd)))
    return pl.pallas_call(
        _st_body,
        grid=(N_BATCH,),
        in_specs=in_specs,
        out_specs=pl.BlockSpec((1, 2, N_NODES, C_HID), lambda b: (b, 0, 0, 0)),
        out_shape=jax.ShapeDtypeStruct((N_BATCH, 2, N_NODES, C_HID),
                                       jnp.float32),
        scratch_shapes=[pltpu.VMEM((N_NODES, N_NODES), jnp.bfloat16)],
    )(*args)


_FC1_KB = 8192
_FC2_RB = 8192
_NK1 = FLATD // _FC1_KB
_NJ2 = FC2_OUT // _FC2_RB


def _fc_body(h_ref, w1_ref, b1_ref, w2_ref, b2_ref, o_ref, hm_scr):
    i = pl.program_id(0)

    @pl.when(i == 0)
    def _():
        hm_scr[...] = jnp.zeros_like(hm_scr)

    @pl.when(i < _NK1)
    def _():
        hm_scr[...] += lax.dot_general(h_ref[...], w1_ref[...],
                                       (((1,), (1,)), ((), ())))

    @pl.when(i >= _NK1)
    def _():
        act = jax.nn.relu(hm_scr[...] + b1_ref[...])
        y = lax.dot_general(act, w2_ref[...], (((1,), (1,)), ((), ())))
        o_ref[...] = jax.nn.sigmoid(y + b2_ref[...]).reshape(
            N_BATCH, _FC2_RB // N_NODES, N_NODES)


def _run_fc(h, w1, b1, w2, b2):
    def clip(i, lo, hi):
        return jnp.minimum(jnp.maximum(i + lo, 0), hi)
    return pl.pallas_call(
        _fc_body,
        grid=(_NK1 + _NJ2,),
        in_specs=[
            pl.BlockSpec((N_BATCH, _FC1_KB), lambda i: (0, clip(i, 0, _NK1 - 1))),
            pl.BlockSpec((FC_H, _FC1_KB), lambda i: (0, clip(i, 0, _NK1 - 1))),
            pl.BlockSpec((1, FC_H), lambda i: (0, 0)),
            pl.BlockSpec((_FC2_RB, FC_H), lambda i: (clip(i, -_NK1, _NJ2 - 1), 0)),
            pl.BlockSpec((1, _FC2_RB), lambda i: (0, clip(i, -_NK1, _NJ2 - 1))),
        ],
        out_specs=pl.BlockSpec((N_BATCH, _FC2_RB // N_NODES, N_NODES),
                               lambda i: (0, clip(i, -_NK1, _NJ2 - 1), 0)),
        out_shape=jax.ShapeDtypeStruct((N_BATCH, N_NODES, N_NODES),
                                       jnp.float32),
        scratch_shapes=[pltpu.VMEM((N_BATCH, FC_H), jnp.float32)],
    )(h, w1, b1[None, :], w2, b2[None, :])


def kernel(x, edge_index, edge_weight, params):
    bp, degp = _sc_build(edge_index.astype(jnp.int32),
                         edge_weight.astype(jnp.float32))
    h = _run_st(x, bp, degp, params)
    return _run_fc(h.reshape(N_BATCH, FLATD), params["fc1_W"], params["fc1_b"],
                   params["fc2_W"], params["fc2_b"])


# speedup vs baseline: 19.3184x; 1.0006x over previous
import functools
import math

import jax
import jax.numpy as jnp
from jax import lax
from jax.experimental import pallas as pl
from jax.experimental.pallas import tpu as pltpu
from jax.experimental.pallas import tpu_sc as plsc

N_NODES = 512
E_EDGES = 8192
N_BATCH = 32
T_WIN = 10
C_IN = 16
C_HID = 64
FC_H = 256
FLATD = 2 * N_NODES * C_HID
FC2_OUT = N_NODES * N_NODES

_NC = 2
_NS = 16
_EPW = E_EDGES // (_NC * _NS)


def _sc_body(ei_ref, ew_ref, bp_out, degp_out,
             rbuf, cbuf, wbuf, idx2, val2, degidx2, zb, shB, shDeg):
    c = lax.axis_index("c")
    s = lax.axis_index("s")
    base = (c * _NS + s) * _EPW

    pltpu.sync_copy(ei_ref.at[0, pl.ds(base, _EPW)], rbuf)
    pltpu.sync_copy(ei_ref.at[1, pl.ds(base, _EPW)], cbuf)
    pltpu.sync_copy(ew_ref.at[pl.ds(base, _EPW)], wbuf)

    for k in range(32):
        zb[pl.ds(k * 16, 16)] = jnp.zeros((16,), jnp.float32)
    for k in range(32):
        pltpu.sync_copy(zb, shB.at[pl.ds((s * 32 + k) * 512, 512)])

    @pl.when(s == 0)
    def _():
        pltpu.sync_copy(zb, shDeg)

    plsc.subcore_barrier()

    for k in range(_EPW // 16):
        sl = pl.ds(k * 16, 16)
        r = rbuf[sl]
        cc = cbuf[sl]
        wv = wbuf[sl]
        wm = jnp.where(r == cc, jnp.zeros((16,), jnp.float32), wv)
        j, kk = divmod(k, 8)
        dsl = pl.ds(kk * 16, 16)
        idx2[j, dsl] = cc * N_NODES + r
        degidx2[j, dsl] = r
        val2[j, dsl] = wm

    for j in range(_EPW // 128):
        pltpu.sync_copy(val2.at[j], shB.at[idx2.at[j]], add=True)
        pltpu.sync_copy(val2.at[j], shDeg.at[degidx2.at[j]], add=True)

    plsc.subcore_barrier()

    seg = (N_NODES * N_NODES) // _NS
    base_b = c * (N_NODES * N_NODES) + s * seg
    pltpu.sync_copy(shB.at[pl.ds(s * seg, seg)], bp_out.at[pl.ds(base_b, seg)])

    @pl.when(s == 0)
    def _():
        pltpu.sync_copy(shDeg, degp_out.at[pl.ds(c * N_NODES, N_NODES)])


def _sc_build(edge_index, edge_weight):
    mesh = plsc.VectorSubcoreMesh(core_axis_name="c", subcore_axis_name="s")
    f = pl.kernel(
        _sc_body,
        out_type=(
            jax.ShapeDtypeStruct((_NC * N_NODES * N_NODES,), jnp.float32),
            jax.ShapeDtypeStruct((_NC * N_NODES,), jnp.float32),
        ),
        mesh=mesh,
        scratch_types=[
            pltpu.VMEM((_EPW,), jnp.int32),
            pltpu.VMEM((_EPW,), jnp.int32),
            pltpu.VMEM((_EPW,), jnp.float32),
            pltpu.VMEM((_EPW // 128, 128), jnp.int32),
            pltpu.VMEM((_EPW // 128, 128), jnp.float32),
            pltpu.VMEM((_EPW // 128, 128), jnp.int32),
            pltpu.VMEM((N_NODES,), jnp.float32),
            pltpu.VMEM_SHARED((N_NODES * N_NODES,), jnp.float32),
            pltpu.VMEM_SHARED((N_NODES,), jnp.float32),
        ],
    )
    return f(edge_index, edge_weight)


def _dotf(a, b):
    return lax.dot_general(a, b, (((1,), (0,)), ((), ())),
                           preferred_element_type=jnp.float32)


def _tconv(hflat, t_in, cin, wc, bc):
    t_out = t_in - 2
    rows = t_out * N_NODES
    cat = jnp.concatenate(
        [hflat[d * N_NODES:d * N_NODES + rows] for d in range(3)], axis=1)
    y = _dotf(cat, wc) + bc
    co = wc.shape[1] // 3
    a, g, c3 = y[:, :co], y[:, co:2 * co], y[:, 2 * co:]
    return jax.nn.relu(a * jax.nn.sigmoid(g) + c3)


def _cheb(hflat, t, a_mat, bd0, bd1, cbt):
    hcat = jnp.concatenate(
        [hflat[i * N_NODES:(i + 1) * N_NODES] for i in range(t)], axis=1)
    p = _bf(_dotf(a_mat, hcat))
    g = _bf(jax.nn.relu(_dotf(hcat, bd0) + _dotf(p, bd1) + cbt))
    return jnp.concatenate(
        [g[:, i * C_HID:(i + 1) * C_HID] for i in range(t)], axis=0)


def _bn_relu(hflat, t, sc, bi):
    h3 = hflat.reshape(t, N_NODES, C_HID)
    h3 = jax.nn.relu(h3 * sc[None] + bi[None])
    return h3.reshape(t * N_NODES, C_HID)


def _bf(v):
    return v.astype(jnp.bfloat16)


def _st_body(x_ref, bp_ref, degc_ref, degr_ref,
             w1c_ref, b1c_ref, bd01_ref, bd11_ref, cb1_ref, w2c_ref, b2c_ref,
             s1_ref, bb1_ref,
             w3c_ref, b3c_ref, bd02_ref, bd12_ref, cb2_ref, w4c_ref, b4c_ref,
             s2_ref, bb2_ref,
             out_ref, a_scr):
    @pl.when(pl.program_id(0) == 0)
    def _():
        dc = degc_ref[0] + degc_ref[1]
        dr = degr_ref[0] + degr_ref[1]
        disc = jnp.where(dc > 0, lax.rsqrt(dc), 0.0)
        disr = jnp.where(dr > 0, lax.rsqrt(dr), 0.0)
        a_scr[...] = _bf(-(disc * disr) * (bp_ref[0] + bp_ref[1]))

    x = _bf(x_ref[0].reshape(T_WIN * N_NODES, C_IN))
    a_mat = a_scr[...]

    h = _bf(_tconv(x, T_WIN, C_IN, w1c_ref[...], b1c_ref[...]))
    h = _cheb(h, 8, a_mat, bd01_ref[...], bd11_ref[...], cb1_ref[...])
    h = _bf(_tconv(h, 8, C_HID, w2c_ref[...], b2c_ref[...]))
    h = _bf(_bn_relu(h, 6, s1_ref[...], bb1_ref[...]))

    h = _bf(_tconv(h, 6, C_HID, w3c_ref[...], b3c_ref[...]))
    h = _cheb(h, 4, a_mat, bd02_ref[...], bd12_ref[...], cb2_ref[...])
    h = _bf(_tconv(h, 4, C_HID, w4c_ref[...], b4c_ref[...]))
    h = _bn_relu(h, 2, s2_ref[...], bb2_ref[...])

    out_ref[0] = h.reshape(2, N_NODES, C_HID).astype(jnp.float32)


def _stack_tconv_w(p, pref):
    ws = []
    bs = []
    for i in (1, 2, 3):
        k = p[pref + "_k%d" % i]
        w = jnp.transpose(k[:, :, 0, :], (2, 1, 0))
        ws.append(w.reshape(-1, k.shape[0]))
        bs.append(p[pref + "_b%d" % i])
    return jnp.concatenate(ws, axis=1), jnp.concatenate(bs)[None, :]


def _run_st(x, bp, degp, p):
    w1c, b1c = _stack_tconv_w(p, "s1t1")
    w2c, b2c = _stack_tconv_w(p, "s1t2")
    w3c, b3c = _stack_tconv_w(p, "s2t1")
    w4c, b4c = _stack_tconv_w(p, "s2t2")
    bnscale = jnp.float32(1.0 / math.sqrt(1.0 + 1e-5))
    eye8 = jnp.eye(8, dtype=jnp.float32)
    eye4 = jnp.eye(4, dtype=jnp.float32)
    args = [
        x,
        bp.reshape(_NC, N_NODES, N_NODES),
        degp.reshape(_NC, N_NODES, 1),
        degp.reshape(_NC, 1, N_NODES),
        _bf(w1c), b1c,
        _bf(jnp.kron(eye8, p["s1_chebW0"].T)),
        _bf(jnp.kron(eye8, p["s1_chebW1"].T)),
        jnp.tile(p["s1_chebb"][None, :], (1, 8)),
        _bf(w2c), b2c,
        (p["bn1_g"] * bnscale)[:, None], p["bn1_b"][:, None],
        _bf(w3c), b3c,
        _bf(jnp.kron(eye4, p["s2_chebW0"].T)),
        _bf(jnp.kron(eye4, p["s2_chebW1"].T)),
        jnp.tile(p["s2_chebb"][None, :], (1, 4)),
        _bf(w4c), b4c,
        (p["bn2_g"] * bnscale)[:, None], p["bn2_b"][:, None],
    ]
    in_specs = [pl.BlockSpec((1,) + x.shape[1:], lambda b: (b, 0, 0, 0))]
    for t in args[1:]:
        nd = t.ndim
        in_specs.append(pl.BlockSpec(t.shape, functools.partial(
            lambda n, b: (0,) * n, nd)))
    return pl.pallas_call(
        _st_body,
        grid=(N_BATCH,),
        in_specs=in_specs,
        out_specs=pl.BlockSpec((1, 2, N_NODES, C_HID), lambda b: (b, 0, 0, 0)),
        out_shape=jax.ShapeDtypeStruct((N_BATCH, 2, N_NODES, C_HID),
                                       jnp.float32),
        scratch_shapes=[pltpu.VMEM((N_NODES, N_NODES), jnp.bfloat16)],
    )(*args)


_FC1_KB = 8192
_FC2_RB = 16384
_NK1 = FLATD // _FC1_KB
_NJ2 = FC2_OUT // _FC2_RB


def _fc_body(h_ref, w1_ref, b1_ref, w2_ref, b2_ref, o_ref, hm_scr):
    i = pl.program_id(0)

    @pl.when(i == 0)
    def _():
        hm_scr[...] = jnp.zeros_like(hm_scr)

    @pl.when(i < _NK1)
    def _():
        hm_scr[...] += lax.dot_general(h_ref[...], w1_ref[...],
                                       (((1,), (1,)), ((), ())))

    @pl.when(i >= _NK1)
    def _():
        act = jax.nn.relu(hm_scr[...] + b1_ref[...])
        y = lax.dot_general(act, w2_ref[...], (((1,), (1,)), ((), ())))
        o_ref[...] = jax.nn.sigmoid(y + b2_ref[...]).reshape(
            N_BATCH, _FC2_RB // N_NODES, N_NODES)


def _run_fc(h, w1, b1, w2, b2):
    def clip(i, lo, hi):
        return jnp.minimum(jnp.maximum(i + lo, 0), hi)
    return pl.pallas_call(
        _fc_body,
        grid=(_NK1 + _NJ2,),
        in_specs=[
            pl.BlockSpec((N_BATCH, _FC1_KB), lambda i: (0, clip(i, 0, _NK1 - 1))),
            pl.BlockSpec((FC_H, _FC1_KB), lambda i: (0, clip(i, 0, _NK1 - 1))),
            pl.BlockSpec((1, FC_H), lambda i: (0, 0)),
            pl.BlockSpec((_FC2_RB, FC_H), lambda i: (clip(i, -_NK1, _NJ2 - 1), 0)),
            pl.BlockSpec((1, _FC2_RB), lambda i: (0, clip(i, -_NK1, _NJ2 - 1))),
        ],
        out_specs=pl.BlockSpec((N_BATCH, _FC2_RB // N_NODES, N_NODES),
                               lambda i: (0, clip(i, -_NK1, _NJ2 - 1), 0)),
        out_shape=jax.ShapeDtypeStruct((N_BATCH, N_NODES, N_NODES),
                                       jnp.float32),
        scratch_shapes=[pltpu.VMEM((N_BATCH, FC_H), jnp.float32)],
    )(h, w1, b1[None, :], w2, b2[None, :])


def kernel(x, edge_index, edge_weight, params):
    bp, degp = _sc_build(edge_index.astype(jnp.int32),
                         edge_weight.astype(jnp.float32))
    h = _run_st(x, bp, degp, params)
    return _run_fc(h.reshape(N_BATCH, FLATD), params["fc1_W"], params["fc1_b"],
                   params["fc2_W"], params["fc2_b"])


# speedup vs baseline: 19.7551x; 1.0226x over previous
import functools
import math

import jax
import jax.numpy as jnp
from jax import lax
from jax.experimental import pallas as pl
from jax.experimental.pallas import tpu as pltpu
from jax.experimental.pallas import tpu_sc as plsc

N_NODES = 512
E_EDGES = 8192
N_BATCH = 32
T_WIN = 10
C_IN = 16
C_HID = 64
FC_H = 256
FLATD = 2 * N_NODES * C_HID
FC2_OUT = N_NODES * N_NODES

_NC = 2
_NS = 16
_EPW = E_EDGES // (_NC * _NS)


def _sc_body(ei_ref, ew_ref, bp_out, degp_out,
             rbuf, cbuf, wbuf, idx2, val2, degidx2, zb, shB, shDeg):
    c = lax.axis_index("c")
    s = lax.axis_index("s")
    base = (c * _NS + s) * _EPW

    pltpu.sync_copy(ei_ref.at[0, pl.ds(base, _EPW)], rbuf)
    pltpu.sync_copy(ei_ref.at[1, pl.ds(base, _EPW)], cbuf)
    pltpu.sync_copy(ew_ref.at[pl.ds(base, _EPW)], wbuf)

    for k in range(32):
        zb[pl.ds(k * 16, 16)] = jnp.zeros((16,), jnp.float32)
    for k in range(32):
        pltpu.sync_copy(zb, shB.at[pl.ds((s * 32 + k) * 512, 512)])

    @pl.when(s == 0)
    def _():
        pltpu.sync_copy(zb, shDeg)

    plsc.subcore_barrier()

    for k in range(_EPW // 16):
        sl = pl.ds(k * 16, 16)
        r = rbuf[sl]
        cc = cbuf[sl]
        wv = wbuf[sl]
        wm = jnp.where(r == cc, jnp.zeros((16,), jnp.float32), wv)
        j, kk = divmod(k, 8)
        dsl = pl.ds(kk * 16, 16)
        idx2[j, dsl] = cc * N_NODES + r
        degidx2[j, dsl] = r
        val2[j, dsl] = wm

    for j in range(_EPW // 128):
        pltpu.sync_copy(val2.at[j], shB.at[idx2.at[j]], add=True)
        pltpu.sync_copy(val2.at[j], shDeg.at[degidx2.at[j]], add=True)

    plsc.subcore_barrier()

    seg = (N_NODES * N_NODES) // _NS
    base_b = c * (N_NODES * N_NODES) + s * seg
    pltpu.sync_copy(shB.at[pl.ds(s * seg, seg)], bp_out.at[pl.ds(base_b, seg)])

    @pl.when(s == 0)
    def _():
        pltpu.sync_copy(shDeg, degp_out.at[pl.ds(c * N_NODES, N_NODES)])


def _sc_build(edge_index, edge_weight):
    mesh = plsc.VectorSubcoreMesh(core_axis_name="c", subcore_axis_name="s")
    f = pl.kernel(
        _sc_body,
        out_type=(
            jax.ShapeDtypeStruct((_NC * N_NODES * N_NODES,), jnp.float32),
            jax.ShapeDtypeStruct((_NC * N_NODES,), jnp.float32),
        ),
        mesh=mesh,
        scratch_types=[
            pltpu.VMEM((_EPW,), jnp.int32),
            pltpu.VMEM((_EPW,), jnp.int32),
            pltpu.VMEM((_EPW,), jnp.float32),
            pltpu.VMEM((_EPW // 128, 128), jnp.int32),
            pltpu.VMEM((_EPW // 128, 128), jnp.float32),
            pltpu.VMEM((_EPW // 128, 128), jnp.int32),
            pltpu.VMEM((N_NODES,), jnp.float32),
            pltpu.VMEM_SHARED((N_NODES * N_NODES,), jnp.float32),
            pltpu.VMEM_SHARED((N_NODES,), jnp.float32),
        ],
    )
    return f(edge_index, edge_weight)


def _dotf(a, b):
    return lax.dot_general(a, b, (((1,), (0,)), ((), ())),
                           preferred_element_type=jnp.float32)


def _tconv(hflat, t_in, cin, wc, bc):
    t_out = t_in - 2
    rows = t_out * N_NODES
    cat = jnp.concatenate(
        [hflat[d * N_NODES:d * N_NODES + rows] for d in range(3)], axis=1)
    y = _dotf(cat, wc) + bc
    co = wc.shape[1] // 3
    a, g, c3 = y[:, :co], y[:, co:2 * co], y[:, 2 * co:]
    return jax.nn.relu(a * jax.nn.sigmoid(g) + c3)


def _cheb(hflat, t, a_mat, bd0, bd1, cbt):
    hcat = jnp.concatenate(
        [hflat[i * N_NODES:(i + 1) * N_NODES] for i in range(t)], axis=1)
    p = _bf(_dotf(a_mat, hcat))
    g = _bf(jax.nn.relu(_dotf(hcat, bd0) + _dotf(p, bd1) + cbt))
    return jnp.concatenate(
        [g[:, i * C_HID:(i + 1) * C_HID] for i in range(t)], axis=0)


def _bn_relu(hflat, t, sc, bi):
    h3 = hflat.reshape(t, N_NODES, C_HID)
    h3 = jax.nn.relu(h3 * sc[None] + bi[None])
    return h3.reshape(t * N_NODES, C_HID)


def _bf(v):
    return v.astype(jnp.bfloat16)


def _st_body(x_ref, bp_ref, degc_ref, degr_ref,
             w1c_ref, b1c_ref, bd01_ref, bd11_ref, cb1_ref, w2c_ref, b2c_ref,
             s1_ref, bb1_ref,
             w3c_ref, b3c_ref, bd02_ref, bd12_ref, cb2_ref, w4c_ref, b4c_ref,
             s2_ref, bb2_ref,
             out_ref, a_scr):
    @pl.when(pl.program_id(0) == 0)
    def _():
        dc = degc_ref[0] + degc_ref[1]
        dr = degr_ref[0] + degr_ref[1]
        disc = jnp.where(dc > 0, lax.rsqrt(dc), 0.0)
        disr = jnp.where(dr > 0, lax.rsqrt(dr), 0.0)
        a_scr[...] = _bf(-(disc * disr) * (bp_ref[0] + bp_ref[1]))

    x = x_ref[0].reshape(T_WIN * N_NODES, C_IN)
    a_mat = a_scr[...]

    h = _bf(_tconv(x, T_WIN, C_IN, w1c_ref[...], b1c_ref[...]))
    h = _cheb(h, 8, a_mat, bd01_ref[...], bd11_ref[...], cb1_ref[...])
    h = _bf(_tconv(h, 8, C_HID, w2c_ref[...], b2c_ref[...]))
    h = _bf(_bn_relu(h, 6, s1_ref[...], bb1_ref[...]))

    h = _bf(_tconv(h, 6, C_HID, w3c_ref[...], b3c_ref[...]))
    h = _cheb(h, 4, a_mat, bd02_ref[...], bd12_ref[...], cb2_ref[...])
    h = _bf(_tconv(h, 4, C_HID, w4c_ref[...], b4c_ref[...]))
    h = _bn_relu(h, 2, s2_ref[...], bb2_ref[...])

    out_ref[0] = h.reshape(2, N_NODES, C_HID).astype(jnp.float32)


def _stack_tconv_w(p, pref):
    ws = []
    bs = []
    for i in (1, 2, 3):
        k = p[pref + "_k%d" % i]
        w = jnp.transpose(k[:, :, 0, :], (2, 1, 0))
        ws.append(w.reshape(-1, k.shape[0]))
        bs.append(p[pref + "_b%d" % i])
    return jnp.concatenate(ws, axis=1), jnp.concatenate(bs)[None, :]


def _run_st(x, bp, degp, p):
    w1c, b1c = _stack_tconv_w(p, "s1t1")
    w2c, b2c = _stack_tconv_w(p, "s1t2")
    w3c, b3c = _stack_tconv_w(p, "s2t1")
    w4c, b4c = _stack_tconv_w(p, "s2t2")
    bnscale = jnp.float32(1.0 / math.sqrt(1.0 + 1e-5))
    eye8 = jnp.eye(8, dtype=jnp.float32)
    eye4 = jnp.eye(4, dtype=jnp.float32)
    args = [
        _bf(x),
        bp.reshape(_NC, N_NODES, N_NODES),
        degp.reshape(_NC, N_NODES, 1),
        degp.reshape(_NC, 1, N_NODES),
        _bf(w1c), b1c,
        _bf(jnp.kron(eye8, p["s1_chebW0"].T)),
        _bf(jnp.kron(eye8, p["s1_chebW1"].T)),
        jnp.tile(p["s1_chebb"][None, :], (1, 8)),
        _bf(w2c), b2c,
        (p["bn1_g"] * bnscale)[:, None], p["bn1_b"][:, None],
        _bf(w3c), b3c,
        _bf(jnp.kron(eye4, p["s2_chebW0"].T)),
        _bf(jnp.kron(eye4, p["s2_chebW1"].T)),
        jnp.tile(p["s2_chebb"][None, :], (1, 4)),
        _bf(w4c), b4c,
        (p["bn2_g"] * bnscale)[:, None], p["bn2_b"][:, None],
    ]
    in_specs = [pl.BlockSpec((1,) + x.shape[1:], lambda b: (b, 0, 0, 0))]
    for t in args[1:]:
        nd = t.ndim
        in_specs.append(pl.BlockSpec(t.shape, functools.partial(
            lambda n, b: (0,) * n, nd)))
    return pl.pallas_call(
        _st_body,
        grid=(N_BATCH,),
        in_specs=in_specs,
        out_specs=pl.BlockSpec((1, 2, N_NODES, C_HID), lambda b: (b, 0, 0, 0)),
        out_shape=jax.ShapeDtypeStruct((N_BATCH, 2, N_NODES, C_HID),
                                       jnp.float32),
        scratch_shapes=[pltpu.VMEM((N_NODES, N_NODES), jnp.bfloat16)],
    )(*args)


_FC1_KB = 8192
_FC2_RB = 16384
_NK1 = FLATD // _FC1_KB
_NJ2 = FC2_OUT // _FC2_RB


def _fc_body(h_ref, w1_ref, b1_ref, w2_ref, b2_ref, o_ref, hm_scr):
    i = pl.program_id(0)

    @pl.when(i == 0)
    def _():
        hm_scr[...] = jnp.zeros_like(hm_scr)

    @pl.when(i < _NK1)
    def _():
        hm_scr[...] += lax.dot_general(h_ref[...], w1_ref[...],
                                       (((1,), (1,)), ((), ())))

    @pl.when(i >= _NK1)
    def _():
        act = jax.nn.relu(hm_scr[...] + b1_ref[...])
        y = lax.dot_general(act, w2_ref[...], (((1,), (1,)), ((), ())))
        o_ref[...] = jax.nn.sigmoid(y + b2_ref[...]).reshape(
            N_BATCH, _FC2_RB // N_NODES, N_NODES)


def _run_fc(h, w1, b1, w2, b2):
    def clip(i, lo, hi):
        return jnp.minimum(jnp.maximum(i + lo, 0), hi)
    return pl.pallas_call(
        _fc_body,
        grid=(_NK1 + _NJ2,),
        in_specs=[
            pl.BlockSpec((N_BATCH, _FC1_KB), lambda i: (0, clip(i, 0, _NK1 - 1))),
            pl.BlockSpec((FC_H, _FC1_KB), lambda i: (0, clip(i, 0, _NK1 - 1))),
            pl.BlockSpec((1, FC_H), lambda i: (0, 0)),
            pl.BlockSpec((_FC2_RB, FC_H), lambda i: (clip(i, -_NK1, _NJ2 - 1), 0)),
            pl.BlockSpec((1, _FC2_RB), lambda i: (0, clip(i, -_NK1, _NJ2 - 1))),
        ],
        out_specs=pl.BlockSpec((N_BATCH, _FC2_RB // N_NODES, N_NODES),
                               lambda i: (0, clip(i, -_NK1, _NJ2 - 1), 0)),
        out_shape=jax.ShapeDtypeStruct((N_BATCH, N_NODES, N_NODES),
                                       jnp.float32),
        scratch_shapes=[pltpu.VMEM((N_BATCH, FC_H), jnp.float32)],
    )(h, w1, b1[None, :], w2, b2[None, :])


def kernel(x, edge_index, edge_weight, params):
    bp, degp = _sc_build(edge_index.astype(jnp.int32),
                         edge_weight.astype(jnp.float32))
    h = _run_st(x, bp, degp, params)
    return _run_fc(h.reshape(N_BATCH, FLATD), params["fc1_W"], params["fc1_b"],
                   params["fc2_W"], params["fc2_b"])


# speedup vs baseline: 22.4525x; 1.1365x over previous
import functools
import math

import jax
import jax.numpy as jnp
from jax import lax
from jax.experimental import pallas as pl
from jax.experimental.pallas import tpu as pltpu
from jax.experimental.pallas import tpu_sc as plsc

N_NODES = 512
E_EDGES = 8192
N_BATCH = 32
T_WIN = 10
C_IN = 16
C_HID = 64
FC_H = 256
FLATD = 2 * N_NODES * C_HID
FC2_OUT = N_NODES * N_NODES

_NC = 2
_NS = 16
_EPW = E_EDGES // (_NC * _NS)


def _sc_body(ei_ref, ew_ref, bp_out, degp_out,
             rbuf, cbuf, wbuf, idx2, val2, degidx2, zb, shB, shDeg):
    c = lax.axis_index("c")
    s = lax.axis_index("s")
    base = (c * _NS + s) * _EPW

    pltpu.sync_copy(ei_ref.at[0, pl.ds(base, _EPW)], rbuf)
    pltpu.sync_copy(ei_ref.at[1, pl.ds(base, _EPW)], cbuf)
    pltpu.sync_copy(ew_ref.at[pl.ds(base, _EPW)], wbuf)

    for k in range(32):
        zb[pl.ds(k * 16, 16)] = jnp.zeros((16,), jnp.float32)
    for k in range(32):
        pltpu.sync_copy(zb, shB.at[pl.ds((s * 32 + k) * 512, 512)])

    @pl.when(s == 0)
    def _():
        pltpu.sync_copy(zb, shDeg)

    plsc.subcore_barrier()

    for k in range(_EPW // 16):
        sl = pl.ds(k * 16, 16)
        r = rbuf[sl]
        cc = cbuf[sl]
        wv = wbuf[sl]
        wm = jnp.where(r == cc, jnp.zeros((16,), jnp.float32), wv)
        j, kk = divmod(k, 8)
        dsl = pl.ds(kk * 16, 16)
        idx2[j, dsl] = cc * N_NODES + r
        degidx2[j, dsl] = r
        val2[j, dsl] = wm

    for j in range(_EPW // 128):
        pltpu.sync_copy(val2.at[j], shB.at[idx2.at[j]], add=True)
        pltpu.sync_copy(val2.at[j], shDeg.at[degidx2.at[j]], add=True)

    plsc.subcore_barrier()

    seg = (N_NODES * N_NODES) // _NS
    base_b = c * (N_NODES * N_NODES) + s * seg
    pltpu.sync_copy(shB.at[pl.ds(s * seg, seg)], bp_out.at[pl.ds(base_b, seg)])

    @pl.when(s == 0)
    def _():
        pltpu.sync_copy(shDeg, degp_out.at[pl.ds(c * N_NODES, N_NODES)])


def _sc_build(edge_index, edge_weight):
    mesh = plsc.VectorSubcoreMesh(core_axis_name="c", subcore_axis_name="s")
    f = pl.kernel(
        _sc_body,
        out_type=(
            jax.ShapeDtypeStruct((_NC * N_NODES * N_NODES,), jnp.float32),
            jax.ShapeDtypeStruct((_NC * N_NODES,), jnp.float32),
        ),
        mesh=mesh,
        scratch_types=[
            pltpu.VMEM((_EPW,), jnp.int32),
            pltpu.VMEM((_EPW,), jnp.int32),
            pltpu.VMEM((_EPW,), jnp.float32),
            pltpu.VMEM((_EPW // 128, 128), jnp.int32),
            pltpu.VMEM((_EPW // 128, 128), jnp.float32),
            pltpu.VMEM((_EPW // 128, 128), jnp.int32),
            pltpu.VMEM((N_NODES,), jnp.float32),
            pltpu.VMEM_SHARED((N_NODES * N_NODES,), jnp.float32),
            pltpu.VMEM_SHARED((N_NODES,), jnp.float32),
        ],
    )
    return f(edge_index, edge_weight)


_ST_NB = 2


def _dotf(a, b):
    return lax.dot_general(a, b, (((1,), (0,)), ((), ())),
                           preferred_element_type=jnp.float32)


def _tconv(hflat, nb, t_in, cin, wc, bc):
    t_out = t_in - 2
    rows = t_out * N_NODES
    taps = []
    for d in range(3):
        sl = [hflat[(b * t_in + d) * N_NODES:(b * t_in + d) * N_NODES + rows]
              for b in range(nb)]
        taps.append(sl[0] if nb == 1 else jnp.concatenate(sl, axis=0))
    cat = jnp.concatenate(taps, axis=1)
    y = _dotf(cat, wc) + bc
    co = wc.shape[1] // 3
    a, g, c3 = y[:, :co], y[:, co:2 * co], y[:, 2 * co:]
    return jax.nn.relu(a * jax.nn.sigmoid(g) + c3)


def _cheb(hflat, nb, t, a_mat, bd0, bd1, cbt):
    w = t * C_HID
    hcats = [jnp.concatenate(
        [hflat[(b * t + i) * N_NODES:(b * t + i + 1) * N_NODES]
         for i in range(t)], axis=1) for b in range(nb)]
    hcat_all = hcats[0] if nb == 1 else jnp.concatenate(hcats, axis=1)
    p = _bf(_dotf(a_mat, hcat_all))
    outs = []
    for b in range(nb):
        g = _bf(jax.nn.relu(
            _dotf(hcats[b], bd0) + _dotf(p[:, b * w:(b + 1) * w], bd1) + cbt))
        outs.extend(g[:, i * C_HID:(i + 1) * C_HID] for i in range(t))
    return jnp.concatenate(outs, axis=0)


def _bn_relu(hflat, nt, sc, bi):
    h3 = hflat.reshape(nt, N_NODES, C_HID)
    h3 = jax.nn.relu(h3 * sc[None] + bi[None])
    return h3.reshape(nt * N_NODES, C_HID)


def _bf(v):
    return v.astype(jnp.bfloat16)


def _st_body(x_ref, bp_ref, degc_ref, degr_ref,
             w1c_ref, b1c_ref, bd01_ref, bd11_ref, cb1_ref, w2c_ref, b2c_ref,
             s1_ref, bb1_ref,
             w3c_ref, b3c_ref, bd02_ref, bd12_ref, cb2_ref, w4c_ref, b4c_ref,
             s2_ref, bb2_ref,
             out_ref, a_scr):
    @pl.when(pl.program_id(0) == 0)
    def _():
        dc = degc_ref[0] + degc_ref[1]
        dr = degr_ref[0] + degr_ref[1]
        disc = jnp.where(dc > 0, lax.rsqrt(dc), 0.0)
        disr = jnp.where(dr > 0, lax.rsqrt(dr), 0.0)
        a_scr[...] = _bf(-(disc * disr) * (bp_ref[0] + bp_ref[1]))

    nb = _ST_NB
    x = x_ref[...].reshape(nb * T_WIN * N_NODES, C_IN)
    a_mat = a_scr[...]

    h = _bf(_tconv(x, nb, T_WIN, C_IN, w1c_ref[...], b1c_ref[...]))
    h = _cheb(h, nb, 8, a_mat, bd01_ref[...], bd11_ref[...], cb1_ref[...])
    h = _bf(_tconv(h, nb, 8, C_HID, w2c_ref[...], b2c_ref[...]))
    h = _bf(_bn_relu(h, nb * 6, s1_ref[...], bb1_ref[...]))

    h = _bf(_tconv(h, nb, 6, C_HID, w3c_ref[...], b3c_ref[...]))
    h = _cheb(h, nb, 4, a_mat, bd02_ref[...], bd12_ref[...], cb2_ref[...])
    h = _bf(_tconv(h, nb, 4, C_HID, w4c_ref[...], b4c_ref[...]))
    h = _bf(_bn_relu(h, nb * 2, s2_ref[...], bb2_ref[...]))

    out_ref[...] = h.reshape(nb, 2, N_NODES, C_HID)


def _stack_tconv_w(p, pref):
    ws = []
    bs = []
    for i in (1, 2, 3):
        k = p[pref + "_k%d" % i]
        w = jnp.transpose(k[:, :, 0, :], (2, 1, 0))
        ws.append(w.reshape(-1, k.shape[0]))
        bs.append(p[pref + "_b%d" % i])
    return jnp.concatenate(ws, axis=1), jnp.concatenate(bs)[None, :]


def _run_st(x, bp, degp, p):
    w1c, b1c = _stack_tconv_w(p, "s1t1")
    w2c, b2c = _stack_tconv_w(p, "s1t2")
    w3c, b3c = _stack_tconv_w(p, "s2t1")
    w4c, b4c = _stack_tconv_w(p, "s2t2")
    bnscale = jnp.float32(1.0 / math.sqrt(1.0 + 1e-5))
    eye8 = jnp.eye(8, dtype=jnp.float32)
    eye4 = jnp.eye(4, dtype=jnp.float32)
    args = [
        _bf(x),
        bp.reshape(_NC, N_NODES, N_NODES),
        degp.reshape(_NC, N_NODES, 1),
        degp.reshape(_NC, 1, N_NODES),
        _bf(w1c), b1c,
        _bf(jnp.kron(eye8, p["s1_chebW0"].T)),
        _bf(jnp.kron(eye8, p["s1_chebW1"].T)),
        jnp.tile(p["s1_chebb"][None, :], (1, 8)),
        _bf(w2c), b2c,
        (p["bn1_g"] * bnscale)[:, None], p["bn1_b"][:, None],
        _bf(w3c), b3c,
        _bf(jnp.kron(eye4, p["s2_chebW0"].T)),
        _bf(jnp.kron(eye4, p["s2_chebW1"].T)),
        jnp.tile(p["s2_chebb"][None, :], (1, 4)),
        _bf(w4c), b4c,
        (p["bn2_g"] * bnscale)[:, None], p["bn2_b"][:, None],
    ]
    in_specs = [pl.BlockSpec((_ST_NB,) + x.shape[1:], lambda b: (b, 0, 0, 0))]
    for t in args[1:]:
        nd = t.ndim
        in_specs.append(pl.BlockSpec(t.shape, functools.partial(
            lambda n, b: (0,) * n, nd)))
    return pl.pallas_call(
        _st_body,
        grid=(N_BATCH // _ST_NB,),
        in_specs=in_specs,
        out_specs=pl.BlockSpec((_ST_NB, 2, N_NODES, C_HID),
                               lambda b: (b, 0, 0, 0)),
        out_shape=jax.ShapeDtypeStruct((N_BATCH, 2, N_NODES, C_HID),
                                       jnp.bfloat16),
        scratch_shapes=[pltpu.VMEM((N_NODES, N_NODES), jnp.bfloat16)],
    )(*args)


_FC1_KB = 8192
_FC2_RB = 16384
_NK1 = FLATD // _FC1_KB
_NJ2 = FC2_OUT // _FC2_RB


def _fc_body(h_ref, w1_ref, b1_ref, w2_ref, b2_ref, o_ref, hm_scr):
    i = pl.program_id(0)

    @pl.when(i == 0)
    def _():
        hm_scr[...] = jnp.zeros_like(hm_scr)

    @pl.when(i < _NK1)
    def _():
        hm_scr[...] += lax.dot_general(
            h_ref[...], w1_ref[...].astype(jnp.bfloat16),
            (((1,), (1,)), ((), ())), preferred_element_type=jnp.float32)

    @pl.when(i >= _NK1)
    def _():
        act = jax.nn.relu(hm_scr[...] + b1_ref[...])
        y = lax.dot_general(act, w2_ref[...], (((1,), (1,)), ((), ())))
        o_ref[...] = jax.nn.sigmoid(y + b2_ref[...]).reshape(
            N_BATCH, _FC2_RB // N_NODES, N_NODES)


def _run_fc(h, w1, b1, w2, b2):
    def clip(i, lo, hi):
        return jnp.minimum(jnp.maximum(i + lo, 0), hi)
    return pl.pallas_call(
        _fc_body,
        grid=(_NK1 + _NJ2,),
        in_specs=[
            pl.BlockSpec((N_BATCH, _FC1_KB), lambda i: (0, clip(i, 0, _NK1 - 1))),
            pl.BlockSpec((FC_H, _FC1_KB), lambda i: (0, clip(i, 0, _NK1 - 1))),
            pl.BlockSpec((1, FC_H), lambda i: (0, 0)),
            pl.BlockSpec((_FC2_RB, FC_H), lambda i: (clip(i, -_NK1, _NJ2 - 1), 0)),
            pl.BlockSpec((1, _FC2_RB), lambda i: (0, clip(i, -_NK1, _NJ2 - 1))),
        ],
        out_specs=pl.BlockSpec((N_BATCH, _FC2_RB // N_NODES, N_NODES),
                               lambda i: (0, clip(i, -_NK1, _NJ2 - 1), 0)),
        out_shape=jax.ShapeDtypeStruct((N_BATCH, N_NODES, N_NODES),
                                       jnp.float32),
        scratch_shapes=[pltpu.VMEM((N_BATCH, FC_H), jnp.float32)],
    )(h, w1, b1[None, :], w2, b2[None, :])


def kernel(x, edge_index, edge_weight, params):
    bp, degp = _sc_build(edge_index.astype(jnp.int32),
                         edge_weight.astype(jnp.float32))
    h = _run_st(x, bp, degp, params)
    return _run_fc(h.reshape(N_BATCH, FLATD), params["fc1_W"], params["fc1_b"],
                   params["fc2_W"], params["fc2_b"])


# speedup vs baseline: 23.0851x; 1.0282x over previous
import functools
import math

import jax
import jax.numpy as jnp
from jax import lax
from jax.experimental import pallas as pl
from jax.experimental.pallas import tpu as pltpu
from jax.experimental.pallas import tpu_sc as plsc

N_NODES = 512
E_EDGES = 8192
N_BATCH = 32
T_WIN = 10
C_IN = 16
C_HID = 64
FC_H = 256
FLATD = 2 * N_NODES * C_HID
FC2_OUT = N_NODES * N_NODES

_NC = 2
_NS = 16
_EPW = E_EDGES // (_NC * _NS)


def _sc_body(ei_ref, ew_ref, bp_out, degp_out,
             rbuf, cbuf, wbuf, idx2, val2, degidx2, zb, shB, shDeg):
    c = lax.axis_index("c")
    s = lax.axis_index("s")
    base = (c * _NS + s) * _EPW

    pltpu.sync_copy(ei_ref.at[0, pl.ds(base, _EPW)], rbuf)
    pltpu.sync_copy(ei_ref.at[1, pl.ds(base, _EPW)], cbuf)
    pltpu.sync_copy(ew_ref.at[pl.ds(base, _EPW)], wbuf)

    for k in range(32):
        zb[pl.ds(k * 16, 16)] = jnp.zeros((16,), jnp.float32)
    for k in range(32):
        pltpu.sync_copy(zb, shB.at[pl.ds((s * 32 + k) * 512, 512)])

    @pl.when(s == 0)
    def _():
        pltpu.sync_copy(zb, shDeg)

    plsc.subcore_barrier()

    for k in range(_EPW // 16):
        sl = pl.ds(k * 16, 16)
        r = rbuf[sl]
        cc = cbuf[sl]
        wv = wbuf[sl]
        wm = jnp.where(r == cc, jnp.zeros((16,), jnp.float32), wv)
        j, kk = divmod(k, 8)
        dsl = pl.ds(kk * 16, 16)
        idx2[j, dsl] = cc * N_NODES + r
        degidx2[j, dsl] = r
        val2[j, dsl] = wm

    for j in range(_EPW // 128):
        pltpu.sync_copy(val2.at[j], shB.at[idx2.at[j]], add=True)
        pltpu.sync_copy(val2.at[j], shDeg.at[degidx2.at[j]], add=True)

    plsc.subcore_barrier()

    seg = (N_NODES * N_NODES) // _NS
    base_b = c * (N_NODES * N_NODES) + s * seg
    pltpu.sync_copy(shB.at[pl.ds(s * seg, seg)], bp_out.at[pl.ds(base_b, seg)])

    @pl.when(s == 0)
    def _():
        pltpu.sync_copy(shDeg, degp_out.at[pl.ds(c * N_NODES, N_NODES)])


def _sc_build(edge_index, edge_weight):
    mesh = plsc.VectorSubcoreMesh(core_axis_name="c", subcore_axis_name="s")
    f = pl.kernel(
        _sc_body,
        out_type=(
            jax.ShapeDtypeStruct((_NC * N_NODES * N_NODES,), jnp.float32),
            jax.ShapeDtypeStruct((_NC * N_NODES,), jnp.float32),
        ),
        mesh=mesh,
        scratch_types=[
            pltpu.VMEM((_EPW,), jnp.int32),
            pltpu.VMEM((_EPW,), jnp.int32),
            pltpu.VMEM((_EPW,), jnp.float32),
            pltpu.VMEM((_EPW // 128, 128), jnp.int32),
            pltpu.VMEM((_EPW // 128, 128), jnp.float32),
            pltpu.VMEM((_EPW // 128, 128), jnp.int32),
            pltpu.VMEM((N_NODES,), jnp.float32),
            pltpu.VMEM_SHARED((N_NODES * N_NODES,), jnp.float32),
            pltpu.VMEM_SHARED((N_NODES,), jnp.float32),
        ],
    )
    return f(edge_index, edge_weight)


_ST_NB = 4


def _dotf(a, b):
    return lax.dot_general(a, b, (((1,), (0,)), ((), ())),
                           preferred_element_type=jnp.float32)


def _tconv(hflat, nb, t_in, cin, wc, bc):
    t_out = t_in - 2
    rows = t_out * N_NODES
    taps = []
    for d in range(3):
        sl = [hflat[(b * t_in + d) * N_NODES:(b * t_in + d) * N_NODES + rows]
              for b in range(nb)]
        taps.append(sl[0] if nb == 1 else jnp.concatenate(sl, axis=0))
    cat = jnp.concatenate(taps, axis=1)
    y = _dotf(cat, wc) + bc
    co = wc.shape[1] // 3
    a, g, c3 = y[:, :co], y[:, co:2 * co], y[:, 2 * co:]
    return jax.nn.relu(a * jax.nn.sigmoid(g) + c3)


def _cheb(hflat, nb, t, a_mat, bd0, bd1, cbt):
    w = t * C_HID
    hcats = [jnp.concatenate(
        [hflat[(b * t + i) * N_NODES:(b * t + i + 1) * N_NODES]
         for i in range(t)], axis=1) for b in range(nb)]
    hcat_all = hcats[0] if nb == 1 else jnp.concatenate(hcats, axis=1)
    p = _bf(_dotf(a_mat, hcat_all))
    outs = []
    for b in range(nb):
        g = _bf(jax.nn.relu(
            _dotf(hcats[b], bd0) + _dotf(p[:, b * w:(b + 1) * w], bd1) + cbt))
        outs.extend(g[:, i * C_HID:(i + 1) * C_HID] for i in range(t))
    return jnp.concatenate(outs, axis=0)


def _bn_relu(hflat, nt, sc, bi):
    h3 = hflat.reshape(nt, N_NODES, C_HID)
    h3 = jax.nn.relu(h3 * sc[None] + bi[None])
    return h3.reshape(nt * N_NODES, C_HID)


def _bf(v):
    return v.astype(jnp.bfloat16)


def _st_body(x_ref, bp_ref, degc_ref, degr_ref,
             w1c_ref, b1c_ref, bd01_ref, bd11_ref, cb1_ref, w2c_ref, b2c_ref,
             s1_ref, bb1_ref,
             w3c_ref, b3c_ref, bd02_ref, bd12_ref, cb2_ref, w4c_ref, b4c_ref,
             s2_ref, bb2_ref,
             out_ref, a_scr):
    @pl.when(pl.program_id(0) == 0)
    def _():
        dc = degc_ref[0] + degc_ref[1]
        dr = degr_ref[0] + degr_ref[1]
        disc = jnp.where(dc > 0, lax.rsqrt(dc), 0.0)
        disr = jnp.where(dr > 0, lax.rsqrt(dr), 0.0)
        a_scr[...] = _bf(-(disc * disr) * (bp_ref[0] + bp_ref[1]))

    nb = _ST_NB
    x = x_ref[...].reshape(nb * T_WIN * N_NODES, C_IN)
    a_mat = a_scr[...]

    h = _bf(_tconv(x, nb, T_WIN, C_IN, w1c_ref[...], b1c_ref[...]))
    h = _cheb(h, nb, 8, a_mat, bd01_ref[...], bd11_ref[...], cb1_ref[...])
    h = _bf(_tconv(h, nb, 8, C_HID, w2c_ref[...], b2c_ref[...]))
    h = _bf(_bn_relu(h, nb * 6, s1_ref[...], bb1_ref[...]))

    h = _bf(_tconv(h, nb, 6, C_HID, w3c_ref[...], b3c_ref[...]))
    h = _cheb(h, nb, 4, a_mat, bd02_ref[...], bd12_ref[...], cb2_ref[...])
    h = _bf(_tconv(h, nb, 4, C_HID, w4c_ref[...], b4c_ref[...]))
    h = _bf(_bn_relu(h, nb * 2, s2_ref[...], bb2_ref[...]))

    out_ref[...] = h.reshape(nb, 2, N_NODES, C_HID)


def _stack_tconv_w(p, pref):
    ws = []
    bs = []
    for i in (1, 2, 3):
        k = p[pref + "_k%d" % i]
        w = jnp.transpose(k[:, :, 0, :], (2, 1, 0))
        ws.append(w.reshape(-1, k.shape[0]))
        bs.append(p[pref + "_b%d" % i])
    return jnp.concatenate(ws, axis=1), jnp.concatenate(bs)[None, :]


def _run_st(x, bp, degp, p):
    w1c, b1c = _stack_tconv_w(p, "s1t1")
    w2c, b2c = _stack_tconv_w(p, "s1t2")
    w3c, b3c = _stack_tconv_w(p, "s2t1")
    w4c, b4c = _stack_tconv_w(p, "s2t2")
    bnscale = jnp.float32(1.0 / math.sqrt(1.0 + 1e-5))
    eye8 = jnp.eye(8, dtype=jnp.float32)
    eye4 = jnp.eye(4, dtype=jnp.float32)
    args = [
        _bf(x),
        bp.reshape(_NC, N_NODES, N_NODES),
        degp.reshape(_NC, N_NODES, 1),
        degp.reshape(_NC, 1, N_NODES),
        _bf(w1c), b1c,
        _bf(jnp.kron(eye8, p["s1_chebW0"].T)),
        _bf(jnp.kron(eye8, p["s1_chebW1"].T)),
        jnp.tile(p["s1_chebb"][None, :], (1, 8)),
        _bf(w2c), b2c,
        (p["bn1_g"] * bnscale)[:, None], p["bn1_b"][:, None],
        _bf(w3c), b3c,
        _bf(jnp.kron(eye4, p["s2_chebW0"].T)),
        _bf(jnp.kron(eye4, p["s2_chebW1"].T)),
        jnp.tile(p["s2_chebb"][None, :], (1, 4)),
        _bf(w4c), b4c,
        (p["bn2_g"] * bnscale)[:, None], p["bn2_b"][:, None],
    ]
    in_specs = [pl.BlockSpec((_ST_NB,) + x.shape[1:], lambda b: (b, 0, 0, 0))]
    for t in args[1:]:
        nd = t.ndim
        in_specs.append(pl.BlockSpec(t.shape, functools.partial(
            lambda n, b: (0,) * n, nd)))
    return pl.pallas_call(
        _st_body,
        grid=(N_BATCH // _ST_NB,),
        in_specs=in_specs,
        out_specs=pl.BlockSpec((_ST_NB, 2, N_NODES, C_HID),
                               lambda b: (b, 0, 0, 0)),
        out_shape=jax.ShapeDtypeStruct((N_BATCH, 2, N_NODES, C_HID),
                                       jnp.bfloat16),
        scratch_shapes=[pltpu.VMEM((N_NODES, N_NODES), jnp.bfloat16)],
    )(*args)


_FC1_KB = 8192
_FC2_RB = 16384
_NK1 = FLATD // _FC1_KB
_NJ2 = FC2_OUT // _FC2_RB


def _fc_body(h_ref, w1_ref, b1_ref, w2_ref, b2_ref, o_ref, hm_scr):
    i = pl.program_id(0)

    @pl.when(i == 0)
    def _():
        hm_scr[...] = jnp.zeros_like(hm_scr)

    @pl.when(i < _NK1)
    def _():
        hm_scr[...] += lax.dot_general(
            h_ref[...], w1_ref[...].astype(jnp.bfloat16),
            (((1,), (1,)), ((), ())), preferred_element_type=jnp.float32)

    @pl.when(i >= _NK1)
    def _():
        act = jax.nn.relu(hm_scr[...] + b1_ref[...])
        y = lax.dot_general(act, w2_ref[...], (((1,), (1,)), ((), ())))
        o_ref[...] = jax.nn.sigmoid(y + b2_ref[...]).reshape(
            N_BATCH, _FC2_RB // N_NODES, N_NODES)


def _run_fc(h, w1, b1, w2, b2):
    def clip(i, lo, hi):
        return jnp.minimum(jnp.maximum(i + lo, 0), hi)
    return pl.pallas_call(
        _fc_body,
        grid=(_NK1 + _NJ2,),
        in_specs=[
            pl.BlockSpec((N_BATCH, _FC1_KB), lambda i: (0, clip(i, 0, _NK1 - 1))),
            pl.BlockSpec((FC_H, _FC1_KB), lambda i: (0, clip(i, 0, _NK1 - 1))),
            pl.BlockSpec((1, FC_H), lambda i: (0, 0)),
            pl.BlockSpec((_FC2_RB, FC_H), lambda i: (clip(i, -_NK1, _NJ2 - 1), 0)),
            pl.BlockSpec((1, _FC2_RB), lambda i: (0, clip(i, -_NK1, _NJ2 - 1))),
        ],
        out_specs=pl.BlockSpec((N_BATCH, _FC2_RB // N_NODES, N_NODES),
                               lambda i: (0, clip(i, -_NK1, _NJ2 - 1), 0)),
        out_shape=jax.ShapeDtypeStruct((N_BATCH, N_NODES, N_NODES),
                                       jnp.float32),
        scratch_shapes=[pltpu.VMEM((N_BATCH, FC_H), jnp.float32)],
    )(h, w1, b1[None, :], w2, b2[None, :])


def kernel(x, edge_index, edge_weight, params):
    bp, degp = _sc_build(edge_index.astype(jnp.int32),
                         edge_weight.astype(jnp.float32))
    h = _run_st(x, bp, degp, params)
    return _run_fc(h.reshape(N_BATCH, FLATD), params["fc1_W"], params["fc1_b"],
                   params["fc2_W"], params["fc2_b"])


# speedup vs baseline: 24.4352x; 1.0585x over previous
import functools
import math

import jax
import jax.numpy as jnp
from jax import lax
from jax.experimental import pallas as pl
from jax.experimental.pallas import tpu as pltpu
from jax.experimental.pallas import tpu_sc as plsc

N_NODES = 512
E_EDGES = 8192
N_BATCH = 32
T_WIN = 10
C_IN = 16
C_HID = 64
FC_H = 256
FLATD = 2 * N_NODES * C_HID
FC2_OUT = N_NODES * N_NODES

_NC = 2
_NS = 16
_EPW = E_EDGES // (_NC * _NS)


def _sc_body(ei_ref, ew_ref, bp_out, degp_out,
             rbuf, cbuf, wbuf, idx2, val2, degidx2, zb, shB, shDeg):
    c = lax.axis_index("c")
    s = lax.axis_index("s")
    base = (c * _NS + s) * _EPW

    pltpu.sync_copy(ei_ref.at[0, pl.ds(base, _EPW)], rbuf)
    pltpu.sync_copy(ei_ref.at[1, pl.ds(base, _EPW)], cbuf)
    pltpu.sync_copy(ew_ref.at[pl.ds(base, _EPW)], wbuf)

    for k in range(32):
        zb[pl.ds(k * 16, 16)] = jnp.zeros((16,), jnp.float32)
    for k in range(32):
        pltpu.sync_copy(zb, shB.at[pl.ds((s * 32 + k) * 512, 512)])

    @pl.when(s == 0)
    def _():
        pltpu.sync_copy(zb, shDeg)

    plsc.subcore_barrier()

    for k in range(_EPW // 16):
        sl = pl.ds(k * 16, 16)
        r = rbuf[sl]
        cc = cbuf[sl]
        wv = wbuf[sl]
        wm = jnp.where(r == cc, jnp.zeros((16,), jnp.float32), wv)
        j, kk = divmod(k, 8)
        dsl = pl.ds(kk * 16, 16)
        idx2[j, dsl] = cc * N_NODES + r
        degidx2[j, dsl] = r
        val2[j, dsl] = wm

    for j in range(_EPW // 128):
        pltpu.sync_copy(val2.at[j], shB.at[idx2.at[j]], add=True)
        pltpu.sync_copy(val2.at[j], shDeg.at[degidx2.at[j]], add=True)

    plsc.subcore_barrier()

    seg = (N_NODES * N_NODES) // _NS
    base_b = c * (N_NODES * N_NODES) + s * seg
    pltpu.sync_copy(shB.at[pl.ds(s * seg, seg)], bp_out.at[pl.ds(base_b, seg)])

    @pl.when(s == 0)
    def _():
        pltpu.sync_copy(shDeg, degp_out.at[pl.ds(c * N_NODES, N_NODES)])


def _sc_build(edge_index, edge_weight):
    mesh = plsc.VectorSubcoreMesh(core_axis_name="c", subcore_axis_name="s")
    f = pl.kernel(
        _sc_body,
        out_type=(
            jax.ShapeDtypeStruct((_NC * N_NODES * N_NODES,), jnp.float32),
            jax.ShapeDtypeStruct((_NC * N_NODES,), jnp.float32),
        ),
        mesh=mesh,
        scratch_types=[
            pltpu.VMEM((_EPW,), jnp.int32),
            pltpu.VMEM((_EPW,), jnp.int32),
            pltpu.VMEM((_EPW,), jnp.float32),
            pltpu.VMEM((_EPW // 128, 128), jnp.int32),
            pltpu.VMEM((_EPW // 128, 128), jnp.float32),
            pltpu.VMEM((_EPW // 128, 128), jnp.int32),
            pltpu.VMEM((N_NODES,), jnp.float32),
            pltpu.VMEM_SHARED((N_NODES * N_NODES,), jnp.float32),
            pltpu.VMEM_SHARED((N_NODES,), jnp.float32),
        ],
    )
    return f(edge_index, edge_weight)


_ST_NB = 4


def _dotf(a, b):
    return lax.dot_general(a, b, (((1,), (0,)), ((), ())),
                           preferred_element_type=jnp.float32)


def _tconv(hflat, nb, t_in, cin, wc, bc):
    t_out = t_in - 2
    rows = t_out * N_NODES
    taps = []
    for d in range(3):
        sl = [hflat[(b * t_in + d) * N_NODES:(b * t_in + d) * N_NODES + rows]
              for b in range(nb)]
        taps.append(sl[0] if nb == 1 else jnp.concatenate(sl, axis=0))
    cat = jnp.concatenate(taps, axis=1)
    y = _dotf(cat, wc) + bc
    co = wc.shape[1] // 3
    a, g, c3 = y[:, :co], y[:, co:2 * co], y[:, 2 * co:]
    return jax.nn.relu(a * jax.nn.sigmoid(g) + c3)


def _cheb(hflat, nb, t, a_mat, bd0, bd1, cbt):
    w = t * C_HID
    hcats = [jnp.concatenate(
        [hflat[(b * t + i) * N_NODES:(b * t + i + 1) * N_NODES]
         for i in range(t)], axis=1) for b in range(nb)]
    hcat_all = hcats[0] if nb == 1 else jnp.concatenate(hcats, axis=1)
    p = _bf(_dotf(a_mat, hcat_all))
    outs = []
    for b in range(nb):
        g = _bf(jax.nn.relu(
            _dotf(hcats[b], bd0) + _dotf(p[:, b * w:(b + 1) * w], bd1) + cbt))
        outs.extend(g[:, i * C_HID:(i + 1) * C_HID] for i in range(t))
    return jnp.concatenate(outs, axis=0)


def _bn_relu(hflat, nt, sc, bi):
    h3 = hflat.reshape(nt, N_NODES, C_HID)
    h3 = jax.nn.relu(h3 * sc[None] + bi[None])
    return h3.reshape(nt * N_NODES, C_HID)


def _bf(v):
    return v.astype(jnp.bfloat16)


def _st_body(x_ref, bp_ref, degc_ref, degr_ref,
             w1c_ref, b1c_ref, bd01_ref, bd11_ref, cb1_ref, w2c_ref, b2c_ref,
             s1_ref, bb1_ref,
             w3c_ref, b3c_ref, bd02_ref, bd12_ref, cb2_ref, w4c_ref, b4c_ref,
             s2_ref, bb2_ref,
             out_ref, a_scr):
    @pl.when(pl.program_id(0) == 0)
    def _():
        dc = degc_ref[0] + degc_ref[1]
        dr = degr_ref[0] + degr_ref[1]
        disc = jnp.where(dc > 0, lax.rsqrt(dc), 0.0)
        disr = jnp.where(dr > 0, lax.rsqrt(dr), 0.0)
        a_scr[...] = _bf(-(disc * disr) * (bp_ref[0] + bp_ref[1]))

    nb = _ST_NB
    x = x_ref[...].reshape(nb * T_WIN * N_NODES, C_IN)
    a_mat = a_scr[...]

    h = _bf(_tconv(x, nb, T_WIN, C_IN, w1c_ref[...], b1c_ref[...]))
    h = _cheb(h, nb, 8, a_mat, bd01_ref[...], bd11_ref[...], cb1_ref[...])
    h = _bf(_tconv(h, nb, 8, C_HID, w2c_ref[...], b2c_ref[...]))
    h = _bf(_bn_relu(h, nb * 6, s1_ref[...], bb1_ref[...]))

    h = _bf(_tconv(h, nb, 6, C_HID, w3c_ref[...], b3c_ref[...]))
    h = _cheb(h, nb, 4, a_mat, bd02_ref[...], bd12_ref[...], cb2_ref[...])
    h = _bf(_tconv(h, nb, 4, C_HID, w4c_ref[...], b4c_ref[...]))
    h = _bf(_bn_relu(h, nb * 2, s2_ref[...], bb2_ref[...]))

    out_ref[...] = h.reshape(nb, 2, N_NODES, C_HID)


def _stack_tconv_w(p, pref):
    k = jnp.stack([p[pref + "_k%d" % i][:, :, 0, :] for i in (1, 2, 3)])
    cout, cin = k.shape[1], k.shape[2]
    wc = jnp.transpose(k, (3, 2, 0, 1)).reshape(3 * cin, 3 * cout)
    b = jnp.stack([p[pref + "_b%d" % i] for i in (1, 2, 3)]).reshape(1, -1)
    return wc, b


def _blockdiag_pair(w0, w1, t):
    eye = jnp.eye(t, dtype=jnp.float32)
    ws = jnp.stack([w0, w1])
    bd = jnp.einsum("ij,eba->eiajb", eye, ws).reshape(2, t * C_HID, t * C_HID)
    return _bf(bd[0]), _bf(bd[1])


def _run_st(x, bp, degp, p):
    w1c, b1c = _stack_tconv_w(p, "s1t1")
    w2c, b2c = _stack_tconv_w(p, "s1t2")
    w3c, b3c = _stack_tconv_w(p, "s2t1")
    w4c, b4c = _stack_tconv_w(p, "s2t2")
    bnscale = jnp.float32(1.0 / math.sqrt(1.0 + 1e-5))
    bd01, bd11 = _blockdiag_pair(p["s1_chebW0"], p["s1_chebW1"], 8)
    bd02, bd12 = _blockdiag_pair(p["s2_chebW0"], p["s2_chebW1"], 4)
    args = [
        _bf(x),
        bp.reshape(_NC, N_NODES, N_NODES),
        degp.reshape(_NC, N_NODES, 1),
        degp.reshape(_NC, 1, N_NODES),
        _bf(w1c), b1c,
        bd01, bd11,
        jnp.tile(p["s1_chebb"][None, :], (1, 8)),
        _bf(w2c), b2c,
        (p["bn1_g"] * bnscale)[:, None], p["bn1_b"][:, None],
        _bf(w3c), b3c,
        bd02, bd12,
        jnp.tile(p["s2_chebb"][None, :], (1, 4)),
        _bf(w4c), b4c,
        (p["bn2_g"] * bnscale)[:, None], p["bn2_b"][:, None],
    ]
    in_specs = [pl.BlockSpec((_ST_NB,) + x.shape[1:], lambda b: (b, 0, 0, 0))]
    for t in args[1:]:
        nd = t.ndim
        in_specs.append(pl.BlockSpec(t.shape, functools.partial(
            lambda n, b: (0,) * n, nd)))
    return pl.pallas_call(
        _st_body,
        grid=(N_BATCH // _ST_NB,),
        in_specs=in_specs,
        out_specs=pl.BlockSpec((_ST_NB, 2, N_NODES, C_HID),
                               lambda b: (b, 0, 0, 0)),
        out_shape=jax.ShapeDtypeStruct((N_BATCH, 2, N_NODES, C_HID),
                                       jnp.bfloat16),
        scratch_shapes=[pltpu.VMEM((N_NODES, N_NODES), jnp.bfloat16)],
    )(*args)


_FC1_KB = 8192
_FC2_RB = 16384
_NK1 = FLATD // _FC1_KB
_NJ2 = FC2_OUT // _FC2_RB


def _fc_body(h_ref, w1_ref, b1_ref, w2_ref, b2_ref, o_ref, hm_scr):
    i = pl.program_id(0)

    @pl.when(i == 0)
    def _():
        hm_scr[...] = jnp.zeros_like(hm_scr)

    @pl.when(i < _NK1)
    def _():
        hm_scr[...] += lax.dot_general(
            h_ref[...], w1_ref[...].astype(jnp.bfloat16),
            (((1,), (1,)), ((), ())), preferred_element_type=jnp.float32)

    @pl.when(i >= _NK1)
    def _():
        act = jax.nn.relu(hm_scr[...] + b1_ref[...])
        y = lax.dot_general(act, w2_ref[...], (((1,), (1,)), ((), ())))
        o_ref[...] = jax.nn.sigmoid(y + b2_ref[...]).reshape(
            N_BATCH, _FC2_RB // N_NODES, N_NODES)


def _run_fc(h, w1, b1, w2, b2):
    def clip(i, lo, hi):
        return jnp.minimum(jnp.maximum(i + lo, 0), hi)
    return pl.pallas_call(
        _fc_body,
        grid=(_NK1 + _NJ2,),
        in_specs=[
            pl.BlockSpec((N_BATCH, _FC1_KB), lambda i: (0, clip(i, 0, _NK1 - 1))),
            pl.BlockSpec((FC_H, _FC1_KB), lambda i: (0, clip(i, 0, _NK1 - 1))),
            pl.BlockSpec((1, FC_H), lambda i: (0, 0)),
            pl.BlockSpec((_FC2_RB, FC_H), lambda i: (clip(i, -_NK1, _NJ2 - 1), 0)),
            pl.BlockSpec((1, _FC2_RB), lambda i: (0, clip(i, -_NK1, _NJ2 - 1))),
        ],
        out_specs=pl.BlockSpec((N_BATCH, _FC2_RB // N_NODES, N_NODES),
                               lambda i: (0, clip(i, -_NK1, _NJ2 - 1), 0)),
        out_shape=jax.ShapeDtypeStruct((N_BATCH, N_NODES, N_NODES),
                                       jnp.float32),
        scratch_shapes=[pltpu.VMEM((N_BATCH, FC_H), jnp.float32)],
    )(h, w1, b1[None, :], w2, b2[None, :])


def kernel(x, edge_index, edge_weight, params):
    bp, degp = _sc_build(edge_index.astype(jnp.int32),
                         edge_weight.astype(jnp.float32))
    h = _run_st(x, bp, degp, params)
    return _run_fc(h.reshape(N_BATCH, FLATD), params["fc1_W"], params["fc1_b"],
                   params["fc2_W"], params["fc2_b"])
